# Initial kernel scaffold; baseline (speedup 1.0000x reference)
import jax
import jax.numpy as jnp
from jax.experimental import pallas as pl


def kernel(x, W_proj, b_proj, basis0, comp0, root0, bias0, basis1, comp1, root1, bias1, edge_index, edge_type):
    raise NotImplementedError("write your pallas kernel here")


# speedup vs baseline: 14.0437x; 14.0437x over previous
import jax
import jax.numpy as jnp
from jax import lax
from jax.experimental import pallas as pl
from jax.experimental.pallas import tpu as pltpu
from jax.experimental.pallas import tpu_sc as plsc

_N = 10000
_D = 128
_R = 4
_NBASES = 8
_Q = 32
_RN = _N * _R
_RN_PAD = 40960
_EB = 128
_TILES = 16
_NBUF = 2
_STRIPE = _RN_PAD // _TILES


def _weights_body(comp0_ref, comp1_ref, basis0_ref, basis1_ref, w0_ref, w1_ref):
    for c_ref, b_ref, w_ref in ((comp0_ref, basis0_ref, w0_ref),
                                (comp1_ref, basis1_ref, w1_ref)):
        b = b_ref[...]
        for r in range(_R):
            acc = c_ref[r, 0] * b[0]
            for k in range(1, _NBASES):
                acc = acc + c_ref[r, k] * b[k]
            w_ref[r] = acc


def _combine_weights(comp0, basis0, comp1, basis1):
    return pl.pallas_call(
        _weights_body,
        in_specs=[
            pl.BlockSpec(memory_space=pltpu.SMEM),
            pl.BlockSpec(memory_space=pltpu.SMEM),
            pl.BlockSpec((_NBASES, _D, _D), lambda: (0, 0, 0)),
            pl.BlockSpec((_NBASES, _D, _D), lambda: (0, 0, 0)),
        ],
        out_specs=[
            pl.BlockSpec((_R, _D, _D), lambda: (0, 0, 0)),
            pl.BlockSpec((_R, _D, _D), lambda: (0, 0, 0)),
        ],
        out_shape=[
            jax.ShapeDtypeStruct((_R, _D, _D), jnp.float32),
            jax.ShapeDtypeStruct((_R, _D, _D), jnp.float32),
        ],
    )(comp0, comp1, basis0, basis1)


_BN = 1000


def _proj_body(x_ref, wt_ref, b_ref, h_ref, hq_ref):
    h = lax.dot_general(x_ref[...], wt_ref[...], (((1,), (0,)), ((), ())),
                        preferred_element_type=jnp.float32)
    h = jnp.maximum(h + b_ref[...], 0.0)
    h_ref[...] = h
    for q in range(4):
        hq_ref[q] = h[:, q * _Q:(q + 1) * _Q]


def _project(x, W_proj, b_proj):
    return pl.pallas_call(
        _proj_body,
        grid=(_N // _BN,),
        in_specs=[
            pl.BlockSpec((_BN, _D), lambda i: (i, 0)),
            pl.BlockSpec((_D, _D), lambda i: (0, 0)),
            pl.BlockSpec((1, _D), lambda i: (0, 0)),
        ],
        out_specs=[
            pl.BlockSpec((_BN, _D), lambda i: (i, 0)),
            pl.BlockSpec((4, _BN, _Q), lambda i: (0, i, 0)),
        ],
        out_shape=[
            jax.ShapeDtypeStruct((_N, _D), jnp.float32),
            jax.ShapeDtypeStruct((4, _N, _Q), jnp.float32),
        ],
    )(x, W_proj.T, b_proj.reshape(1, _D))


def _make_conv_body(relu, split):
    def body(h_ref, a_ref, rc_ref, w_ref, root_ref, bias_ref, out_ref,
             *maybe_hq):
        acc = lax.dot_general(h_ref[...], root_ref[...],
                              (((1,), (0,)), ((), ())),
                              preferred_element_type=jnp.float32)
        w = w_ref[...]
        rc = rc_ref[...]
        for q in range(4):
            aq = a_ref[q]
            for r in range(_R):
                col = aq[:, r * _Q:(r + 1) * _Q] * rc[:, r:r + 1]
                acc = acc + lax.dot_general(
                    col, w[r, q * _Q:(q + 1) * _Q, :],
                    (((1,), (0,)), ((), ())),
                    preferred_element_type=jnp.float32)
        acc = acc + bias_ref[...]
        if relu:
            acc = jnp.maximum(acc, 0.0)
        out_ref[...] = acc
        if split:
            hq_ref = maybe_hq[0]
            for q in range(4):
                hq_ref[q] = acc[:, q * _Q:(q + 1) * _Q]
    return body


def _conv_combine(h, a, rc, w, root, bias, relu, split):
    out_specs = [pl.BlockSpec((_BN, _D), lambda i: (i, 0))]
    out_shape = [jax.ShapeDtypeStruct((_N, _D), jnp.float32)]
    if split:
        out_specs.append(pl.BlockSpec((4, _BN, _Q), lambda i: (0, i, 0)))
        out_shape.append(jax.ShapeDtypeStruct((4, _N, _Q), jnp.float32))
    return pl.pallas_call(
        _make_conv_body(relu, split),
        grid=(_N // _BN,),
        in_specs=[
            pl.BlockSpec((_BN, _D), lambda i: (i, 0)),
            pl.BlockSpec((4, _BN, _D), lambda i: (0, i, 0)),
            pl.BlockSpec((_BN, _R), lambda i: (i, 0)),
            pl.BlockSpec((_R, _D, _D), lambda i: (0, 0, 0)),
            pl.BlockSpec((_D, _D), lambda i: (0, 0)),
            pl.BlockSpec((1, _D), lambda i: (0, 0)),
        ],
        out_specs=out_specs,
        out_shape=out_shape,
    )(h, a, rc, w, root, bias.reshape(1, _D))


_CH = 16


def _make_edge_body(nch, with_counts):
    def body(hs_ref, src4_ref, srow_ref, agg_ref, *rest):
        if with_counts:
            recip_ref, rest = rest[0], rest[1:]
        (agg_sh, cnt_sh, zbuf, rows_a, rows_b, sidx, ridx, ones_v, rbuf,
         sem_a, sem_b) = rest
        rows_bufs = (rows_a, rows_b)
        sems = (sem_a, sem_b)
        cid = lax.axis_index("c")
        sid = lax.axis_index("s")

        z16 = jnp.zeros((16,), jnp.float32)

        def _zero_zbuf(i, _):
            zbuf[i, pl.ds(0, 16)] = z16
            zbuf[i, pl.ds(16, 16)] = z16
            return 0
        lax.fori_loop(0, _EB, _zero_zbuf, 0)

        def _zero_rbuf(i, _):
            rbuf[pl.ds(i * 16, 16)] = z16
            return 0
        lax.fori_loop(0, _STRIPE // 16, _zero_rbuf, 0)

        if with_counts:
            one16 = jnp.ones((16,), jnp.float32)
            for i in range(_EB // 16):
                ones_v[pl.ds(i * 16, 16)] = one16
            pltpu.sync_copy(rbuf, cnt_sh.at[pl.ds(sid * _STRIPE, _STRIPE)])

        for p in range(2):
            q = cid + 2 * p

            def _zero_agg(j, _):
                pltpu.sync_copy(
                    zbuf, agg_sh.at[pl.ds(sid * _STRIPE + j * _EB, _EB), :])
                return 0
            lax.fori_loop(0, _STRIPE // _EB, _zero_agg, 0)
            plsc.subcore_barrier()

            def _chunk(c, _):
                pltpu.sync_copy(src4_ref.at[q * _TILES + sid, c], sidx)
                pltpu.sync_copy(srow_ref.at[sid, c], ridx)
                for k in range(_NBUF):
                    pltpu.async_copy(hs_ref.at[sidx.at[k]], rows_bufs[k],
                                     sems[k])

                def _group(g, _):
                    for k in range(_NBUF):
                        i = g * _NBUF + k
                        pltpu.make_async_copy(hs_ref.at[sidx.at[0]],
                                              rows_bufs[k], sems[k]).wait()
                        pltpu.sync_copy(rows_bufs[k], agg_sh.at[ridx.at[i]],
                                        add=True)
                        if with_counts and p == 0:
                            @pl.when(cid == 0)
                            def _():
                                pltpu.sync_copy(ones_v,
                                                cnt_sh.at[ridx.at[i]],
                                                add=True)
                        nxt = jnp.minimum(i + _NBUF, _CH - 1)
                        pltpu.async_copy(hs_ref.at[sidx.at[nxt]],
                                         rows_bufs[k], sems[k])
                    return 0
                lax.fori_loop(0, _CH // _NBUF, _group, 0)
                for k in range(_NBUF):
                    pltpu.make_async_copy(hs_ref.at[sidx.at[0]], rows_bufs[k],
                                          sems[k]).wait()
                return 0
            lax.fori_loop(0, nch, _chunk, 0)
            plsc.subcore_barrier()

            pltpu.sync_copy(
                agg_sh.at[pl.ds(sid * _STRIPE, _STRIPE), :],
                agg_ref.at[pl.ds(q * _RN_PAD + sid * _STRIPE, _STRIPE), :])
            plsc.subcore_barrier()

        if with_counts:
            @pl.when(cid == 0)
            def _():
                pltpu.sync_copy(cnt_sh.at[pl.ds(sid * _STRIPE, _STRIPE)],
                                rbuf)

                def _recip(j, _):
                    c = rbuf[pl.ds(j * 16, 16)]
                    rbuf[pl.ds(j * 16, 16)] = 1.0 / jnp.maximum(c, 1.0)
                    return 0
                lax.fori_loop(0, _STRIPE // 16, _recip, 0)
                pltpu.sync_copy(rbuf,
                                recip_ref.at[pl.ds(sid * _STRIPE, _STRIPE)])

    return body


def _edge_pass(hs, src4, srow3, with_counts):
    nch = src4.shape[1]
    out_type = [jax.ShapeDtypeStruct((4 * _RN_PAD, _Q), jnp.float32)]
    if with_counts:
        out_type.append(jax.ShapeDtypeStruct((_RN_PAD,), jnp.float32))
    scratch = [
        pltpu.VMEM_SHARED((_RN_PAD, _Q), jnp.float32),
        pltpu.VMEM_SHARED((_RN_PAD,), jnp.float32),
        pltpu.VMEM((_EB, _Q), jnp.float32),
        pltpu.VMEM((_EB, _Q), jnp.float32),
        pltpu.VMEM((_EB, _Q), jnp.float32),
        pltpu.VMEM((_CH, _EB), jnp.int32),
        pltpu.VMEM((_CH, _EB), jnp.int32),
        pltpu.VMEM((_EB,), jnp.float32),
        pltpu.VMEM((_STRIPE,), jnp.float32),
        pltpu.SemaphoreType.DMA,
        pltpu.SemaphoreType.DMA,
    ]
    mesh = plsc.VectorSubcoreMesh(core_axis_name="c", subcore_axis_name="s",
                                  num_cores=2, num_subcores=_TILES)
    fn = pl.kernel(
        _make_edge_body(nch, with_counts),
        out_type=tuple(out_type),
        mesh=mesh,
        scratch_types=scratch,
        compiler_params=pltpu.CompilerParams(use_tc_tiling_on_sc=False),
    )
    return fn(hs, src4, srow3)


def _edge_indices(edge_index, edge_type):
    e = edge_index.shape[1]
    bpt = -(-e // (_TILES * _EB))
    if bpt % _CH:
        bpt += _CH - bpt % _CH
    nch = bpt // _CH
    e_pad = _TILES * _EB * bpt
    pad = e_pad - e
    src = edge_index[0].astype(jnp.int32)
    srow = (edge_index[1] * _R + edge_type).astype(jnp.int32)
    pad_ar = jnp.arange(pad, dtype=jnp.int32)
    src_p = jnp.concatenate([src, pad_ar % _N])
    srow_p = jnp.concatenate([srow, _RN + pad_ar % (_RN_PAD - _RN)])
    srow3 = srow_p.reshape(_TILES, nch, _CH, _EB)
    src4 = (src_p[None, :] + (jnp.arange(4, dtype=jnp.int32) * _N)[:, None])
    src4 = src4.reshape(4 * _TILES, nch, _CH, _EB)
    return src4, srow3


def kernel(x, W_proj, b_proj, basis0, comp0, root0, bias0,
           basis1, comp1, root1, bias1, edge_index, edge_type):
    w0, w1 = _combine_weights(comp0, basis0, comp1, basis1)
    h, hq0 = _project(x, W_proj, b_proj)
    src4, srow3 = _edge_indices(edge_index, edge_type)

    agg0, recip = _edge_pass(hq0.reshape(4 * _N, _Q), src4, srow3,
                             with_counts=True)
    a0 = agg0.reshape(4, _RN_PAD // _R, _D)
    rc = recip.reshape(_RN_PAD // _R, _R)
    x1, hq1 = _conv_combine(h, a0, rc, w0, root0, bias0,
                            relu=True, split=True)

    agg1 = _edge_pass(hq1.reshape(4 * _N, _Q), src4, srow3,
                      with_counts=False)[0]
    a1 = agg1.reshape(4, _RN_PAD // _R, _D)
    out = _conv_combine(x1, a1, rc, w1, root1, bias1,
                        relu=False, split=False)[0]
    return out, h, h


# speedup vs baseline: 16.3040x; 1.1609x over previous
import jax
import jax.numpy as jnp
from jax import lax
from jax.experimental import pallas as pl
from jax.experimental.pallas import tpu as pltpu
from jax.experimental.pallas import tpu_sc as plsc

_N = 10000
_D = 128
_R = 4
_NBASES = 8
_Q = 32
_RN = _N * _R
_RN_PAD = 40960
_EB = 128
_TILES = 16
_NBUF = 2
_STRIPE = _RN_PAD // _TILES


def _weights_body(comp0_ref, comp1_ref, basis0_ref, basis1_ref, w0_ref, w1_ref):
    for c_ref, b_ref, w_ref in ((comp0_ref, basis0_ref, w0_ref),
                                (comp1_ref, basis1_ref, w1_ref)):
        b = b_ref[...]
        for r in range(_R):
            acc = c_ref[r, 0] * b[0]
            for k in range(1, _NBASES):
                acc = acc + c_ref[r, k] * b[k]
            w_ref[r] = acc


def _combine_weights(comp0, basis0, comp1, basis1):
    return pl.pallas_call(
        _weights_body,
        in_specs=[
            pl.BlockSpec(memory_space=pltpu.SMEM),
            pl.BlockSpec(memory_space=pltpu.SMEM),
            pl.BlockSpec((_NBASES, _D, _D), lambda: (0, 0, 0)),
            pl.BlockSpec((_NBASES, _D, _D), lambda: (0, 0, 0)),
        ],
        out_specs=[
            pl.BlockSpec((_R, _D, _D), lambda: (0, 0, 0)),
            pl.BlockSpec((_R, _D, _D), lambda: (0, 0, 0)),
        ],
        out_shape=[
            jax.ShapeDtypeStruct((_R, _D, _D), jnp.float32),
            jax.ShapeDtypeStruct((_R, _D, _D), jnp.float32),
        ],
    )(comp0, comp1, basis0, basis1)


_BN = 1000


def _proj_body(x_ref, wt_ref, b_ref, h_ref, hq_ref):
    h = lax.dot_general(x_ref[...], wt_ref[...], (((1,), (0,)), ((), ())),
                        preferred_element_type=jnp.float32)
    h = jnp.maximum(h + b_ref[...], 0.0)
    h_ref[...] = h
    for q in range(4):
        hq_ref[q] = h[:, q * _Q:(q + 1) * _Q]


def _project(x, W_proj, b_proj):
    return pl.pallas_call(
        _proj_body,
        grid=(_N // _BN,),
        in_specs=[
            pl.BlockSpec((_BN, _D), lambda i: (i, 0)),
            pl.BlockSpec((_D, _D), lambda i: (0, 0)),
            pl.BlockSpec((1, _D), lambda i: (0, 0)),
        ],
        out_specs=[
            pl.BlockSpec((_BN, _D), lambda i: (i, 0)),
            pl.BlockSpec((4, _BN, _Q), lambda i: (0, i, 0)),
        ],
        out_shape=[
            jax.ShapeDtypeStruct((_N, _D), jnp.float32),
            jax.ShapeDtypeStruct((4, _NP, _Q), jnp.float32),
        ],
    )(x, W_proj.T, b_proj.reshape(1, _D))


def _make_conv_body(relu, split):
    def body(h_ref, a_ref, rc_ref, w_ref, root_ref, bias_ref, out_ref,
             *maybe_hq):
        acc = lax.dot_general(h_ref[...], root_ref[...],
                              (((1,), (0,)), ((), ())),
                              preferred_element_type=jnp.float32)
        w = w_ref[...]
        rc = rc_ref[...]
        for q in range(4):
            aq = a_ref[q]
            for r in range(_R):
                col = aq[:, r * _Q:(r + 1) * _Q] * rc[:, r:r + 1]
                acc = acc + lax.dot_general(
                    col, w[r, q * _Q:(q + 1) * _Q, :],
                    (((1,), (0,)), ((), ())),
                    preferred_element_type=jnp.float32)
        acc = acc + bias_ref[...]
        if relu:
            acc = jnp.maximum(acc, 0.0)
        out_ref[...] = acc
        if split:
            hq_ref = maybe_hq[0]
            for q in range(4):
                hq_ref[q] = acc[:, q * _Q:(q + 1) * _Q]
    return body


def _conv_combine(h, a, rc, w, root, bias, relu, split):
    out_specs = [pl.BlockSpec((_BN, _D), lambda i: (i, 0))]
    out_shape = [jax.ShapeDtypeStruct((_N, _D), jnp.float32)]
    if split:
        out_specs.append(pl.BlockSpec((4, _BN, _Q), lambda i: (0, i, 0)))
        out_shape.append(jax.ShapeDtypeStruct((4, _NP, _Q), jnp.float32))
    return pl.pallas_call(
        _make_conv_body(relu, split),
        grid=(_N // _BN,),
        in_specs=[
            pl.BlockSpec((_BN, _D), lambda i: (i, 0)),
            pl.BlockSpec((4, _BN, _D), lambda i: (0, i, 0)),
            pl.BlockSpec((_BN, _R), lambda i: (i, 0)),
            pl.BlockSpec((_R, _D, _D), lambda i: (0, 0, 0)),
            pl.BlockSpec((_D, _D), lambda i: (0, 0)),
            pl.BlockSpec((1, _D), lambda i: (0, 0)),
        ],
        out_specs=out_specs,
        out_shape=out_shape,
    )(h, a, rc, w, root, bias.reshape(1, _D))


_CH = 16
_NP = 10240


def _make_edge_body(nch, with_counts):
    def body(hs_ref, src_ref, srow_ref, agg_ref, *rest):
        if with_counts:
            recip_ref, rest = rest[0], rest[1:]
        (agg_sh, cnt_sh, table_sh, zbuf, rows_a, rows_b, sidx, ridx, ones_v,
         rbuf, sem_a, sem_b) = rest
        rows_bufs = (rows_a, rows_b)
        sems = (sem_a, sem_b)
        cid = lax.axis_index("c")
        sid = lax.axis_index("s")

        z16 = jnp.zeros((16,), jnp.float32)

        def _zero_zbuf(i, _):
            zbuf[i, pl.ds(0, 16)] = z16
            zbuf[i, pl.ds(16, 16)] = z16
            return 0
        lax.fori_loop(0, _EB, _zero_zbuf, 0)

        def _zero_rbuf(i, _):
            rbuf[pl.ds(i * 16, 16)] = z16
            return 0
        lax.fori_loop(0, _STRIPE // 16, _zero_rbuf, 0)

        if with_counts:
            one16 = jnp.ones((16,), jnp.float32)
            for i in range(_EB // 16):
                ones_v[pl.ds(i * 16, 16)] = one16
            pltpu.sync_copy(rbuf, cnt_sh.at[pl.ds(sid * _STRIPE, _STRIPE)])

        tstripe = _NP // _TILES
        for p in range(2):
            q = cid + 2 * p

            pltpu.sync_copy(
                hs_ref.at[pl.ds(q * _NP + sid * tstripe, tstripe), :],
                table_sh.at[pl.ds(sid * tstripe, tstripe), :])

            def _zero_agg(j, _):
                pltpu.sync_copy(
                    zbuf, agg_sh.at[pl.ds(sid * _STRIPE + j * _EB, _EB), :])
                return 0
            lax.fori_loop(0, _STRIPE // _EB, _zero_agg, 0)
            plsc.subcore_barrier()

            def _chunk(c, _):
                pltpu.sync_copy(src_ref.at[sid, c], sidx)
                pltpu.sync_copy(srow_ref.at[sid, c], ridx)
                for k in range(_NBUF):
                    pltpu.async_copy(table_sh.at[sidx.at[k]], rows_bufs[k],
                                     sems[k])

                def _group(g, _):
                    for k in range(_NBUF):
                        i = g * _NBUF + k
                        pltpu.make_async_copy(table_sh.at[sidx.at[0]],
                                              rows_bufs[k], sems[k]).wait()
                        pltpu.sync_copy(rows_bufs[k], agg_sh.at[ridx.at[i]],
                                        add=True)
                        if with_counts and p == 0:
                            @pl.when(cid == 0)
                            def _():
                                pltpu.sync_copy(ones_v,
                                                cnt_sh.at[ridx.at[i]],
                                                add=True)
                        nxt = jnp.minimum(i + _NBUF, _CH - 1)
                        pltpu.async_copy(table_sh.at[sidx.at[nxt]],
                                         rows_bufs[k], sems[k])
                    return 0
                lax.fori_loop(0, _CH // _NBUF, _group, 0)
                for k in range(_NBUF):
                    pltpu.make_async_copy(table_sh.at[sidx.at[0]],
                                          rows_bufs[k], sems[k]).wait()
                return 0
            lax.fori_loop(0, nch, _chunk, 0)
            plsc.subcore_barrier()

            pltpu.sync_copy(
                agg_sh.at[pl.ds(sid * _STRIPE, _STRIPE), :],
                agg_ref.at[pl.ds(q * _RN_PAD + sid * _STRIPE, _STRIPE), :])
            plsc.subcore_barrier()

        if with_counts:
            @pl.when(cid == 0)
            def _():
                pltpu.sync_copy(cnt_sh.at[pl.ds(sid * _STRIPE, _STRIPE)],
                                rbuf)

                def _recip(j, _):
                    c = rbuf[pl.ds(j * 16, 16)]
                    rbuf[pl.ds(j * 16, 16)] = 1.0 / jnp.maximum(c, 1.0)
                    return 0
                lax.fori_loop(0, _STRIPE // 16, _recip, 0)
                pltpu.sync_copy(rbuf,
                                recip_ref.at[pl.ds(sid * _STRIPE, _STRIPE)])

    return body


def _edge_pass(hs, src4, srow3, with_counts):
    nch = src4.shape[1]
    out_type = [jax.ShapeDtypeStruct((4 * _RN_PAD, _Q), jnp.float32)]
    if with_counts:
        out_type.append(jax.ShapeDtypeStruct((_RN_PAD,), jnp.float32))
    scratch = [
        pltpu.VMEM_SHARED((_RN_PAD, _Q), jnp.float32),
        pltpu.VMEM_SHARED((_RN_PAD,), jnp.float32),
        pltpu.VMEM_SHARED((_NP, _Q), jnp.float32),
        pltpu.VMEM((_EB, _Q), jnp.float32),
        pltpu.VMEM((_EB, _Q), jnp.float32),
        pltpu.VMEM((_EB, _Q), jnp.float32),
        pltpu.VMEM((_CH, _EB), jnp.int32),
        pltpu.VMEM((_CH, _EB), jnp.int32),
        pltpu.VMEM((_EB,), jnp.float32),
        pltpu.VMEM((_STRIPE,), jnp.float32),
        pltpu.SemaphoreType.DMA,
        pltpu.SemaphoreType.DMA,
    ]
    mesh = plsc.VectorSubcoreMesh(core_axis_name="c", subcore_axis_name="s",
                                  num_cores=2, num_subcores=_TILES)
    fn = pl.kernel(
        _make_edge_body(nch, with_counts),
        out_type=tuple(out_type),
        mesh=mesh,
        scratch_types=scratch,
        compiler_params=pltpu.CompilerParams(use_tc_tiling_on_sc=False),
    )
    return fn(hs, src4, srow3)


def _edge_indices(edge_index, edge_type):
    e = edge_index.shape[1]
    bpt = -(-e // (_TILES * _EB))
    if bpt % _CH:
        bpt += _CH - bpt % _CH
    nch = bpt // _CH
    e_pad = _TILES * _EB * bpt
    pad = e_pad - e
    src = edge_index[0].astype(jnp.int32)
    srow = (edge_index[1] * _R + edge_type).astype(jnp.int32)
    pad_ar = jnp.arange(pad, dtype=jnp.int32)
    src_p = jnp.concatenate([src, pad_ar % _N])
    srow_p = jnp.concatenate([srow, _RN + pad_ar % (_RN_PAD - _RN)])
    srow3 = srow_p.reshape(_TILES, nch, _CH, _EB)
    src3 = src_p.reshape(_TILES, nch, _CH, _EB)
    return src3, srow3


def kernel(x, W_proj, b_proj, basis0, comp0, root0, bias0,
           basis1, comp1, root1, bias1, edge_index, edge_type):
    w0, w1 = _combine_weights(comp0, basis0, comp1, basis1)
    h, hq0 = _project(x, W_proj, b_proj)
    src3, srow3 = _edge_indices(edge_index, edge_type)

    agg0, recip = _edge_pass(hq0.reshape(4 * _NP, _Q), src3, srow3,
                             with_counts=True)
    a0 = agg0.reshape(4, _RN_PAD // _R, _D)
    rc = recip.reshape(_RN_PAD // _R, _R)
    x1, hq1 = _conv_combine(h, a0, rc, w0, root0, bias0,
                            relu=True, split=True)

    agg1 = _edge_pass(hq1.reshape(4 * _NP, _Q), src3, srow3,
                      with_counts=False)[0]
    a1 = agg1.reshape(4, _RN_PAD // _R, _D)
    out = _conv_combine(x1, a1, rc, w1, root1, bias1,
                        relu=False, split=False)[0]
    return out, h, h


# speedup vs baseline: 19.9624x; 1.2244x over previous
import jax
import jax.numpy as jnp
from jax import lax
from jax.experimental import pallas as pl
from jax.experimental.pallas import tpu as pltpu
from jax.experimental.pallas import tpu_sc as plsc

_N = 10000
_D = 128
_R = 4
_NBASES = 8
_Q = 32
_RN = _N * _R
_RN_PAD = 40960
_EB = 128
_TILES = 16
_NBUF = 4
_ZR = 64
_STRIPE = _RN_PAD // _TILES


def _weights_body(comp0_ref, comp1_ref, basis0_ref, basis1_ref, w0_ref, w1_ref):
    for c_ref, b_ref, w_ref in ((comp0_ref, basis0_ref, w0_ref),
                                (comp1_ref, basis1_ref, w1_ref)):
        b = b_ref[...]
        for r in range(_R):
            acc = c_ref[r, 0] * b[0]
            for k in range(1, _NBASES):
                acc = acc + c_ref[r, k] * b[k]
            for q in range(4):
                w_ref[pl.ds(q * _D + r * _Q, _Q), :] = \
                    acc[q * _Q:(q + 1) * _Q, :]


def _combine_weights(comp0, basis0, comp1, basis1):
    return pl.pallas_call(
        _weights_body,
        in_specs=[
            pl.BlockSpec(memory_space=pltpu.SMEM),
            pl.BlockSpec(memory_space=pltpu.SMEM),
            pl.BlockSpec((_NBASES, _D, _D), lambda: (0, 0, 0)),
            pl.BlockSpec((_NBASES, _D, _D), lambda: (0, 0, 0)),
        ],
        out_specs=[
            pl.BlockSpec((4 * _D, _D), lambda: (0, 0)),
            pl.BlockSpec((4 * _D, _D), lambda: (0, 0)),
        ],
        out_shape=[
            jax.ShapeDtypeStruct((4 * _D, _D), jnp.float32),
            jax.ShapeDtypeStruct((4 * _D, _D), jnp.float32),
        ],
    )(comp0, comp1, basis0, basis1)


_BN = 1000


def _proj_body(x_ref, wt_ref, b_ref, h_ref):
    h = lax.dot_general(x_ref[...], wt_ref[...], (((1,), (0,)), ((), ())),
                        preferred_element_type=jnp.float32)
    h_ref[...] = jnp.maximum(h + b_ref[...], 0.0)


def _project(x, W_proj, b_proj):
    return pl.pallas_call(
        _proj_body,
        grid=(_N // _BN,),
        in_specs=[
            pl.BlockSpec((_BN, _D), lambda i: (i, 0)),
            pl.BlockSpec((_D, _D), lambda i: (0, 0)),
            pl.BlockSpec((1, _D), lambda i: (0, 0)),
        ],
        out_specs=pl.BlockSpec((_BN, _D), lambda i: (i, 0)),
        out_shape=jax.ShapeDtypeStruct((_NP, _D), jnp.float32),
    )(x, W_proj.T, b_proj.reshape(1, _D))


def _make_conv_body(relu):
    def body(h_ref, a_ref, c0_ref, c1_ref, w_ref, root_ref, bias_ref,
             out_ref):
        acc = lax.dot_general(h_ref[...], root_ref[...],
                              (((1,), (0,)), ((), ())),
                              preferred_element_type=jnp.float32)
        rc = 1.0 / jnp.maximum(c0_ref[...] + c1_ref[...], 1.0)
        rcx = jnp.concatenate(
            [jnp.broadcast_to(rc[:, r:r + 1], (_BN, _Q)) for r in range(_R)],
            axis=1)
        cat = jnp.concatenate([a_ref[q] * rcx for q in range(4)], axis=1)
        acc = acc + lax.dot_general(cat, w_ref[...],
                                    (((1,), (0,)), ((), ())),
                                    preferred_element_type=jnp.float32)
        acc = acc + bias_ref[...]
        if relu:
            acc = jnp.maximum(acc, 0.0)
        out_ref[...] = acc
    return body


def _conv_combine(h, a, cnt0, cnt1, w, root, bias, relu, out_rows):
    return pl.pallas_call(
        _make_conv_body(relu),
        grid=(_N // _BN,),
        in_specs=[
            pl.BlockSpec((_BN, _D), lambda i: (i, 0)),
            pl.BlockSpec((4, _BN, _D), lambda i: (0, i, 0)),
            pl.BlockSpec((_BN, _R), lambda i: (i, 0)),
            pl.BlockSpec((_BN, _R), lambda i: (i, 0)),
            pl.BlockSpec((4 * _D, _D), lambda i: (0, 0)),
            pl.BlockSpec((_D, _D), lambda i: (0, 0)),
            pl.BlockSpec((1, _D), lambda i: (0, 0)),
        ],
        out_specs=pl.BlockSpec((_BN, _D), lambda i: (i, 0)),
        out_shape=jax.ShapeDtypeStruct((out_rows, _D), jnp.float32),
    )(h, a, cnt0, cnt1, w, root, bias.reshape(1, _D))


_CH = 16
_NP = 10240


def _make_edge_body(nch, with_counts):
    def body(hs_ref, src_ref, srow_ref, agg_ref, *rest):
        if with_counts:
            cnt0_ref, cnt1_ref, rest = rest[0], rest[1], rest[2:]
        (agg_sh, cnt_sh, table_sh, zbuf, r0, r1, r2, r3, sidx, ridx, ones_v,
         rbuf, g0, g1, g2, g3, s0, s1, s2, s3, csem) = rest
        rows_bufs = (r0, r1, r2, r3)
        gsems = (g0, g1, g2, g3)
        ssems = (s0, s1, s2, s3)
        cid = lax.axis_index("c")
        sid = lax.axis_index("s")

        z16 = jnp.zeros((16,), jnp.float32)

        def _zero_zbuf(i, _):
            zbuf[i, pl.ds(0, 16)] = z16
            zbuf[i, pl.ds(16, 16)] = z16
            return 0
        lax.fori_loop(0, _ZR, _zero_zbuf, 0)

        def _zero_rbuf(i, _):
            rbuf[pl.ds(i * 16, 16)] = z16
            return 0
        lax.fori_loop(0, _STRIPE // 16, _zero_rbuf, 0)

        if with_counts:
            one16 = jnp.ones((16,), jnp.float32)
            for i in range(_EB // 16):
                ones_v[pl.ds(i * 16, 16)] = one16
            pltpu.sync_copy(rbuf, cnt_sh.at[pl.ds(sid * _STRIPE, _STRIPE)])

        tstripe = _NP // _TILES
        half = nch // 2
        for p in range(2):
            q = cid + 2 * p

            pltpu.sync_copy(
                hs_ref.at[pl.ds(sid * tstripe, tstripe), pl.ds(q * _Q, _Q)],
                table_sh.at[pl.ds(sid * tstripe, tstripe), :])

            def _zero_agg(j, _):
                pltpu.sync_copy(
                    zbuf, agg_sh.at[pl.ds(sid * _STRIPE + j * _ZR, _ZR), :])
                return 0
            lax.fori_loop(0, _STRIPE // _ZR, _zero_agg, 0)
            plsc.subcore_barrier()

            counting = with_counts and p == 0

            def _chunk(c, _):
                pltpu.sync_copy(src_ref.at[sid, c], sidx)
                pltpu.sync_copy(srow_ref.at[sid, c], ridx)
                for k in range(2):
                    pltpu.async_copy(table_sh.at[sidx.at[k]], rows_bufs[k],
                                     gsems[k])
                for j in range(_CH):
                    k = j % _NBUF
                    if j + 2 < _CH:
                        k2 = (j + 2) % _NBUF
                        if j - 2 >= 0:
                            pltpu.make_async_copy(
                                rows_bufs[k2], agg_sh.at[ridx.at[0]],
                                ssems[k2]).wait()
                        pltpu.async_copy(table_sh.at[sidx.at[j + 2]],
                                         rows_bufs[k2], gsems[k2])
                    pltpu.make_async_copy(table_sh.at[sidx.at[0]],
                                          rows_bufs[k], gsems[k]).wait()
                    pltpu.async_copy(rows_bufs[k], agg_sh.at[ridx.at[j]],
                                     ssems[k], add=True)
                    if counting:
                        @pl.when((cid == 0) == (c < half))
                        def _():
                            pltpu.async_copy(ones_v, cnt_sh.at[ridx.at[j]],
                                             csem, add=True)
                for k in range(_NBUF):
                    pltpu.make_async_copy(rows_bufs[k], agg_sh.at[ridx.at[0]],
                                          ssems[k]).wait()
                if counting:
                    @pl.when((cid == 0) == (c < half))
                    def _():
                        for j in range(_CH):
                            pltpu.make_async_copy(
                                ones_v, cnt_sh.at[ridx.at[0]], csem).wait()
                return 0
            lax.fori_loop(0, nch, _chunk, 0)
            plsc.subcore_barrier()

            pltpu.sync_copy(
                agg_sh.at[pl.ds(sid * _STRIPE, _STRIPE), :],
                agg_ref.at[pl.ds(q * _RN_PAD + sid * _STRIPE, _STRIPE), :])
            if with_counts and p == 0:
                @pl.when(cid == 0)
                def _():
                    pltpu.sync_copy(cnt_sh.at[pl.ds(sid * _STRIPE, _STRIPE)],
                                    cnt0_ref.at[pl.ds(sid * _STRIPE,
                                                      _STRIPE)])

                @pl.when(cid == 1)
                def _():
                    pltpu.sync_copy(cnt_sh.at[pl.ds(sid * _STRIPE, _STRIPE)],
                                    cnt1_ref.at[pl.ds(sid * _STRIPE,
                                                      _STRIPE)])
            plsc.subcore_barrier()

    return body


def _edge_pass(hs, src4, srow3, with_counts):
    nch = src4.shape[1]
    out_type = [jax.ShapeDtypeStruct((4 * _RN_PAD, _Q), jnp.float32)]
    if with_counts:
        out_type.append(jax.ShapeDtypeStruct((_RN_PAD,), jnp.float32))
        out_type.append(jax.ShapeDtypeStruct((_RN_PAD,), jnp.float32))
    scratch = (
        [
            pltpu.VMEM_SHARED((_RN_PAD, _Q), jnp.float32),
            pltpu.VMEM_SHARED((_RN_PAD,), jnp.float32),
            pltpu.VMEM_SHARED((_NP, _Q), jnp.float32),
            pltpu.VMEM((_ZR, _Q), jnp.float32),
        ]
        + [pltpu.VMEM((_EB, _Q), jnp.float32)] * _NBUF
        + [
            pltpu.VMEM((_CH, _EB), jnp.int32),
            pltpu.VMEM((_CH, _EB), jnp.int32),
            pltpu.VMEM((_EB,), jnp.float32),
            pltpu.VMEM((_STRIPE,), jnp.float32),
        ]
        + [pltpu.SemaphoreType.DMA] * (2 * _NBUF + 1)
    )
    mesh = plsc.VectorSubcoreMesh(core_axis_name="c", subcore_axis_name="s",
                                  num_cores=2, num_subcores=_TILES)
    fn = pl.kernel(
        _make_edge_body(nch, with_counts),
        out_type=tuple(out_type),
        mesh=mesh,
        scratch_types=scratch,
        compiler_params=pltpu.CompilerParams(use_tc_tiling_on_sc=False),
    )
    return fn(hs, src4, srow3)


def _edge_indices(edge_index, edge_type):
    e = edge_index.shape[1]
    bpt = -(-e // (_TILES * _EB))
    if bpt % _CH:
        bpt += _CH - bpt % _CH
    nch = bpt // _CH
    e_pad = _TILES * _EB * bpt
    pad = e_pad - e
    src = edge_index[0].astype(jnp.int32)
    srow = (edge_index[1] * _R + edge_type).astype(jnp.int32)
    pad_ar = jnp.arange(pad, dtype=jnp.int32)
    src_p = jnp.concatenate([src, pad_ar % _N])
    srow_p = jnp.concatenate([srow, _RN + pad_ar % (_RN_PAD - _RN)])
    srow3 = srow_p.reshape(_TILES, nch, _CH, _EB)
    src3 = src_p.reshape(_TILES, nch, _CH, _EB)
    return src3, srow3


def kernel(x, W_proj, b_proj, basis0, comp0, root0, bias0,
           basis1, comp1, root1, bias1, edge_index, edge_type):
    w0, w1 = _combine_weights(comp0, basis0, comp1, basis1)
    h = _project(x, W_proj, b_proj)
    src3, srow3 = _edge_indices(edge_index, edge_type)

    agg0, cnt0, cnt1 = _edge_pass(h, src3, srow3, with_counts=True)
    a0 = agg0.reshape(4, _RN_PAD // _R, _D)
    c0 = cnt0.reshape(_RN_PAD // _R, _R)
    c1 = cnt1.reshape(_RN_PAD // _R, _R)
    x1 = _conv_combine(h, a0, c0, c1, w0, root0, bias0,
                       relu=True, out_rows=_NP)

    agg1 = _edge_pass(x1, src3, srow3, with_counts=False)[0]
    a1 = agg1.reshape(4, _RN_PAD // _R, _D)
    out = _conv_combine(x1, a1, c0, c1, w1, root1, bias1,
                        relu=False, out_rows=_N)
    hn = h[:_N]
    return out, hn, hn


# speedup vs baseline: 20.3311x; 1.0185x over previous
import jax
import jax.numpy as jnp
from jax import lax
from jax.experimental import pallas as pl
from jax.experimental.pallas import tpu as pltpu
from jax.experimental.pallas import tpu_sc as plsc

_N = 10000
_D = 128
_R = 4
_NBASES = 8
_Q = 32
_RN = _N * _R
_RN_PAD = 40960
_EB = 128
_TILES = 16
_NBUF = 4
_ZR = 64
_STRIPE = _RN_PAD // _TILES


def _weights_body(comp0_ref, comp1_ref, basis0_ref, basis1_ref, w0_ref, w1_ref):
    for c_ref, b_ref, w_ref in ((comp0_ref, basis0_ref, w0_ref),
                                (comp1_ref, basis1_ref, w1_ref)):
        b = b_ref[...]
        for r in range(_R):
            acc = c_ref[r, 0] * b[0]
            for k in range(1, _NBASES):
                acc = acc + c_ref[r, k] * b[k]
            for q in range(4):
                w_ref[pl.ds(q * _D + r * _Q, _Q), :] = \
                    acc[q * _Q:(q + 1) * _Q, :]


def _combine_weights(comp0, basis0, comp1, basis1):
    return pl.pallas_call(
        _weights_body,
        in_specs=[
            pl.BlockSpec(memory_space=pltpu.SMEM),
            pl.BlockSpec(memory_space=pltpu.SMEM),
            pl.BlockSpec((_NBASES, _D, _D), lambda: (0, 0, 0)),
            pl.BlockSpec((_NBASES, _D, _D), lambda: (0, 0, 0)),
        ],
        out_specs=[
            pl.BlockSpec((4 * _D, _D), lambda: (0, 0)),
            pl.BlockSpec((4 * _D, _D), lambda: (0, 0)),
        ],
        out_shape=[
            jax.ShapeDtypeStruct((4 * _D, _D), jnp.float32),
            jax.ShapeDtypeStruct((4 * _D, _D), jnp.float32),
        ],
    )(comp0, comp1, basis0, basis1)


_BN = 1000


def _proj_body(x_ref, wt_ref, b_ref, h_ref):
    h = lax.dot_general(x_ref[...], wt_ref[...], (((1,), (0,)), ((), ())),
                        preferred_element_type=jnp.float32)
    h_ref[...] = jnp.maximum(h + b_ref[...], 0.0)


def _project(x, W_proj, b_proj):
    return pl.pallas_call(
        _proj_body,
        grid=(_N // _BN,),
        in_specs=[
            pl.BlockSpec((_BN, _D), lambda i: (i, 0)),
            pl.BlockSpec((_D, _D), lambda i: (0, 0)),
            pl.BlockSpec((1, _D), lambda i: (0, 0)),
        ],
        out_specs=pl.BlockSpec((_BN, _D), lambda i: (i, 0)),
        out_shape=jax.ShapeDtypeStruct((_N, _D), jnp.float32),
    )(x, W_proj.T, b_proj.reshape(1, _D))


_IB = 128


def _make_idx_body(e):
    def body(src_ref, dst_ref, et_ref, srco_ref, srowo_ref):
        i = pl.program_id(0)
        g = (i * (_IB * 128)
             + lax.broadcasted_iota(jnp.int32, (_IB, 128), 0) * 128
             + lax.broadcasted_iota(jnp.int32, (_IB, 128), 1))
        mask = g < e
        srco_ref[...] = jnp.where(mask, src_ref[...], g % _N)
        srowo_ref[...] = jnp.where(mask, dst_ref[...] * _R + et_ref[...],
                                   _RN + g % (_RN_PAD - _RN))
    return body


def _edge_indices(edge_index, edge_type):
    e = edge_index.shape[1]
    assert e % 128 == 0
    rows = e // 128
    bpt = -(-e // (_TILES * _EB))
    if bpt % _CH:
        bpt += _CH - bpt % _CH
    nch = bpt // _CH
    rows_pad = _TILES * bpt
    grid = rows_pad // _IB
    src_v = edge_index[0].reshape(rows, 128)
    dst_v = edge_index[1].reshape(rows, 128)
    et_v = edge_type.reshape(rows, 128).astype(jnp.int32)
    srco, srowo = pl.pallas_call(
        _make_idx_body(e),
        grid=(grid,),
        in_specs=[
            pl.BlockSpec((_IB, 128), lambda i: (i, 0)),
            pl.BlockSpec((_IB, 128), lambda i: (i, 0)),
            pl.BlockSpec((_IB, 128), lambda i: (i, 0)),
        ],
        out_specs=[
            pl.BlockSpec((_IB, 128), lambda i: (i, 0)),
            pl.BlockSpec((_IB, 128), lambda i: (i, 0)),
        ],
        out_shape=[
            jax.ShapeDtypeStruct((rows_pad, 128), jnp.int32),
            jax.ShapeDtypeStruct((rows_pad, 128), jnp.int32),
        ],
    )(src_v, dst_v, et_v)
    src3 = srco.reshape(_TILES, nch, _CH, _EB)
    srow3 = srowo.reshape(_TILES, nch, _CH, _EB)
    return src3, srow3


def _make_conv_body(relu):
    def body(h_ref, a_ref, c0_ref, c1_ref, w_ref, root_ref, bias_ref,
             out_ref):
        acc = lax.dot_general(h_ref[...], root_ref[...],
                              (((1,), (0,)), ((), ())),
                              preferred_element_type=jnp.float32)
        rc = 1.0 / jnp.maximum(c0_ref[...] + c1_ref[...], 1.0)
        rcx = jnp.concatenate(
            [jnp.broadcast_to(rc[:, r:r + 1], (_BN, _Q)) for r in range(_R)],
            axis=1)
        cat = jnp.concatenate([a_ref[q] * rcx for q in range(4)], axis=1)
        acc = acc + lax.dot_general(cat, w_ref[...],
                                    (((1,), (0,)), ((), ())),
                                    preferred_element_type=jnp.float32)
        acc = acc + bias_ref[...]
        if relu:
            acc = jnp.maximum(acc, 0.0)
        out_ref[...] = acc
    return body


def _conv_combine(h, a, cnt0, cnt1, w, root, bias, relu, out_rows):
    return pl.pallas_call(
        _make_conv_body(relu),
        grid=(_N // _BN,),
        in_specs=[
            pl.BlockSpec((_BN, _D), lambda i: (i, 0)),
            pl.BlockSpec((4, _BN, _D), lambda i: (0, i, 0)),
            pl.BlockSpec((_BN, _R), lambda i: (i, 0)),
            pl.BlockSpec((_BN, _R), lambda i: (i, 0)),
            pl.BlockSpec((4 * _D, _D), lambda i: (0, 0)),
            pl.BlockSpec((_D, _D), lambda i: (0, 0)),
            pl.BlockSpec((1, _D), lambda i: (0, 0)),
        ],
        out_specs=pl.BlockSpec((_BN, _D), lambda i: (i, 0)),
        out_shape=jax.ShapeDtypeStruct((out_rows, _D), jnp.float32),
    )(h, a, cnt0, cnt1, w, root, bias.reshape(1, _D))


_CH = 16
_NP = 10240


def _make_edge_body(nch, with_counts):
    def body(hs_ref, src_ref, srow_ref, agg_ref, *rest):
        if with_counts:
            cnt0_ref, cnt1_ref, rest = rest[0], rest[1], rest[2:]
        (agg_sh, cnt_sh, table_sh, zbuf, r0, r1, r2, r3, sidx, ridx, ones_v,
         rbuf, g0, g1, g2, g3, s0, s1, s2, s3, csem) = rest
        rows_bufs = (r0, r1, r2, r3)
        gsems = (g0, g1, g2, g3)
        ssems = (s0, s1, s2, s3)
        cid = lax.axis_index("c")
        sid = lax.axis_index("s")

        z16 = jnp.zeros((16,), jnp.float32)

        def _zero_zbuf(i, _):
            zbuf[i, pl.ds(0, 16)] = z16
            zbuf[i, pl.ds(16, 16)] = z16
            return 0
        lax.fori_loop(0, _ZR, _zero_zbuf, 0)

        def _zero_rbuf(i, _):
            rbuf[pl.ds(i * 16, 16)] = z16
            return 0
        lax.fori_loop(0, _STRIPE // 16, _zero_rbuf, 0)

        if with_counts:
            one16 = jnp.ones((16,), jnp.float32)
            for i in range(_EB // 16):
                ones_v[pl.ds(i * 16, 16)] = one16
            pltpu.sync_copy(rbuf, cnt_sh.at[pl.ds(sid * _STRIPE, _STRIPE)])

        ts0 = 632
        ts15 = _N - 15 * ts0
        half = nch // 2
        for p in range(2):
            q = cid + 2 * p

            @pl.when(sid != 15)
            def _():
                pltpu.async_copy(
                    hs_ref.at[pl.ds(sid * ts0, ts0), pl.ds(q * _Q, _Q)],
                    table_sh.at[pl.ds(sid * ts0, ts0), :], csem)

            @pl.when(sid == 15)
            def _():
                pltpu.async_copy(
                    hs_ref.at[pl.ds(15 * ts0, ts15), pl.ds(q * _Q, _Q)],
                    table_sh.at[pl.ds(15 * ts0, ts15), :], csem)

            nz = _STRIPE // _ZR
            for j in range(nz):
                k = j % _NBUF
                if j >= _NBUF:
                    pltpu.make_async_copy(
                        zbuf, agg_sh.at[pl.ds(0, _ZR), :], gsems[k]).wait()
                pltpu.async_copy(
                    zbuf, agg_sh.at[pl.ds(sid * _STRIPE + j * _ZR, _ZR), :],
                    gsems[k])
            for k in range(_NBUF):
                pltpu.make_async_copy(zbuf, agg_sh.at[pl.ds(0, _ZR), :],
                                      gsems[k]).wait()

            @pl.when(sid != 15)
            def _():
                pltpu.make_async_copy(
                    hs_ref.at[pl.ds(sid * ts0, ts0), pl.ds(q * _Q, _Q)],
                    table_sh.at[pl.ds(sid * ts0, ts0), :], csem).wait()

            @pl.when(sid == 15)
            def _():
                pltpu.make_async_copy(
                    hs_ref.at[pl.ds(15 * ts0, ts15), pl.ds(q * _Q, _Q)],
                    table_sh.at[pl.ds(15 * ts0, ts15), :], csem).wait()
            plsc.subcore_barrier()

            counting = with_counts and p == 0

            def _chunk(c, _):
                pltpu.sync_copy(src_ref.at[sid, c], sidx)
                pltpu.sync_copy(srow_ref.at[sid, c], ridx)
                for k in range(2):
                    pltpu.async_copy(table_sh.at[sidx.at[k]], rows_bufs[k],
                                     gsems[k])
                for j in range(_CH):
                    k = j % _NBUF
                    if j + 2 < _CH:
                        k2 = (j + 2) % _NBUF
                        if j - 2 >= 0:
                            pltpu.make_async_copy(
                                rows_bufs[k2], agg_sh.at[ridx.at[0]],
                                ssems[k2]).wait()
                        pltpu.async_copy(table_sh.at[sidx.at[j + 2]],
                                         rows_bufs[k2], gsems[k2])
                    pltpu.make_async_copy(table_sh.at[sidx.at[0]],
                                          rows_bufs[k], gsems[k]).wait()
                    pltpu.async_copy(rows_bufs[k], agg_sh.at[ridx.at[j]],
                                     ssems[k], add=True)
                    if counting:
                        @pl.when((cid == 0) == (c < half))
                        def _():
                            pltpu.async_copy(ones_v, cnt_sh.at[ridx.at[j]],
                                             csem, add=True)
                for k in range(_NBUF):
                    pltpu.make_async_copy(rows_bufs[k], agg_sh.at[ridx.at[0]],
                                          ssems[k]).wait()
                if counting:
                    @pl.when((cid == 0) == (c < half))
                    def _():
                        for j in range(_CH):
                            pltpu.make_async_copy(
                                ones_v, cnt_sh.at[ridx.at[0]], csem).wait()
                return 0
            lax.fori_loop(0, nch, _chunk, 0)
            plsc.subcore_barrier()

            pltpu.sync_copy(
                agg_sh.at[pl.ds(sid * _STRIPE, _STRIPE), :],
                agg_ref.at[pl.ds(q * _RN_PAD + sid * _STRIPE, _STRIPE), :])
            if with_counts and p == 0:
                @pl.when(cid == 0)
                def _():
                    pltpu.sync_copy(cnt_sh.at[pl.ds(sid * _STRIPE, _STRIPE)],
                                    cnt0_ref.at[pl.ds(sid * _STRIPE,
                                                      _STRIPE)])

                @pl.when(cid == 1)
                def _():
                    pltpu.sync_copy(cnt_sh.at[pl.ds(sid * _STRIPE, _STRIPE)],
                                    cnt1_ref.at[pl.ds(sid * _STRIPE,
                                                      _STRIPE)])
            plsc.subcore_barrier()

    return body


def _edge_pass(hs, src4, srow3, with_counts):
    nch = src4.shape[1]
    out_type = [jax.ShapeDtypeStruct((4 * _RN_PAD, _Q), jnp.float32)]
    if with_counts:
        out_type.append(jax.ShapeDtypeStruct((_RN_PAD,), jnp.float32))
        out_type.append(jax.ShapeDtypeStruct((_RN_PAD,), jnp.float32))
    scratch = (
        [
            pltpu.VMEM_SHARED((_RN_PAD, _Q), jnp.float32),
            pltpu.VMEM_SHARED((_RN_PAD,), jnp.float32),
            pltpu.VMEM_SHARED((_NP, _Q), jnp.float32),
            pltpu.VMEM((_ZR, _Q), jnp.float32),
        ]
        + [pltpu.VMEM((_EB, _Q), jnp.float32)] * _NBUF
        + [
            pltpu.VMEM((_CH, _EB), jnp.int32),
            pltpu.VMEM((_CH, _EB), jnp.int32),
            pltpu.VMEM((_EB,), jnp.float32),
            pltpu.VMEM((_STRIPE,), jnp.float32),
        ]
        + [pltpu.SemaphoreType.DMA] * (2 * _NBUF + 1)
    )
    mesh = plsc.VectorSubcoreMesh(core_axis_name="c", subcore_axis_name="s",
                                  num_cores=2, num_subcores=_TILES)
    fn = pl.kernel(
        _make_edge_body(nch, with_counts),
        out_type=tuple(out_type),
        mesh=mesh,
        scratch_types=scratch,
        compiler_params=pltpu.CompilerParams(use_tc_tiling_on_sc=False),
    )
    return fn(hs, src4, srow3)


def kernel(x, W_proj, b_proj, basis0, comp0, root0, bias0,
           basis1, comp1, root1, bias1, edge_index, edge_type):
    w0, w1 = _combine_weights(comp0, basis0, comp1, basis1)
    h = _project(x, W_proj, b_proj)
    src3, srow3 = _edge_indices(edge_index, edge_type)

    agg0, cnt0, cnt1 = _edge_pass(h, src3, srow3, with_counts=True)
    a0 = agg0.reshape(4, _RN_PAD // _R, _D)
    c0 = cnt0.reshape(_RN_PAD // _R, _R)
    c1 = cnt1.reshape(_RN_PAD // _R, _R)
    x1 = _conv_combine(h, a0, c0, c1, w0, root0, bias0,
                       relu=True, out_rows=_N)

    agg1 = _edge_pass(x1, src3, srow3, with_counts=False)[0]
    a1 = agg1.reshape(4, _RN_PAD // _R, _D)
    out = _conv_combine(x1, a1, c0, c1, w1, root1, bias1,
                        relu=False, out_rows=_N)
    return out, h, h


# speedup vs baseline: 20.9621x; 1.0310x over previous
import jax
import jax.numpy as jnp
from jax import lax
from jax.experimental import pallas as pl
from jax.experimental.pallas import tpu as pltpu
from jax.experimental.pallas import tpu_sc as plsc

_N = 10000
_D = 128
_R = 4
_NBASES = 8
_Q = 32
_RN = _N * _R
_RN_PAD = 40960
_EB = 128
_TILES = 16
_NBUF = 4
_ZR = 64
_STRIPE = _RN_PAD // _TILES


def _weights_body(comp0_ref, comp1_ref, basis0_ref, basis1_ref, w0_ref, w1_ref):
    for c_ref, b_ref, w_ref in ((comp0_ref, basis0_ref, w0_ref),
                                (comp1_ref, basis1_ref, w1_ref)):
        b = b_ref[...]
        for r in range(_R):
            acc = c_ref[r, 0] * b[0]
            for k in range(1, _NBASES):
                acc = acc + c_ref[r, k] * b[k]
            for q in range(4):
                w_ref[pl.ds(q * _D + r * _Q, _Q), :] = \
                    acc[q * _Q:(q + 1) * _Q, :]


def _combine_weights(comp0, basis0, comp1, basis1):
    return pl.pallas_call(
        _weights_body,
        in_specs=[
            pl.BlockSpec(memory_space=pltpu.SMEM),
            pl.BlockSpec(memory_space=pltpu.SMEM),
            pl.BlockSpec((_NBASES, _D, _D), lambda: (0, 0, 0)),
            pl.BlockSpec((_NBASES, _D, _D), lambda: (0, 0, 0)),
        ],
        out_specs=[
            pl.BlockSpec((4 * _D, _D), lambda: (0, 0)),
            pl.BlockSpec((4 * _D, _D), lambda: (0, 0)),
        ],
        out_shape=[
            jax.ShapeDtypeStruct((4 * _D, _D), jnp.float32),
            jax.ShapeDtypeStruct((4 * _D, _D), jnp.float32),
        ],
    )(comp0, comp1, basis0, basis1)


_BN = 1000


def _proj_body(x_ref, wt_ref, b_ref, h_ref):
    h = lax.dot_general(x_ref[...], wt_ref[...], (((1,), (0,)), ((), ())),
                        preferred_element_type=jnp.float32)
    h_ref[...] = jnp.maximum(h + b_ref[...], 0.0)


def _project(x, W_proj, b_proj):
    return pl.pallas_call(
        _proj_body,
        grid=(_N // _BN,),
        in_specs=[
            pl.BlockSpec((_BN, _D), lambda i: (i, 0)),
            pl.BlockSpec((_D, _D), lambda i: (0, 0)),
            pl.BlockSpec((1, _D), lambda i: (0, 0)),
        ],
        out_specs=pl.BlockSpec((_BN, _D), lambda i: (i, 0)),
        out_shape=jax.ShapeDtypeStruct((_N, _D), jnp.float32),
    )(x, W_proj.T, b_proj.reshape(1, _D))


_IB = 128


def _make_idx_body(e):
    def body(ei_ref, et_ref, srco_ref, srowo_ref):
        i = pl.program_id(0)
        g = (i * (_IB * 128)
             + lax.broadcasted_iota(jnp.int32, (_IB, 128), 0) * 128
             + lax.broadcasted_iota(jnp.int32, (_IB, 128), 1))
        mask = g < e
        src = ei_ref[0].reshape(_IB, 128)
        dst = ei_ref[1].reshape(_IB, 128)
        srco_ref[...] = jnp.where(mask, src, g % _N)
        srowo_ref[...] = jnp.where(mask, dst * _R + et_ref[...],
                                   _RN + g % (_RN_PAD - _RN))
    return body


def _edge_indices(edge_index, edge_type):
    e = edge_index.shape[1]
    assert e % 128 == 0
    rows = e // 128
    bpt = -(-e // (_TILES * _EB))
    if bpt % _CH:
        bpt += _CH - bpt % _CH
    nch = bpt // _CH
    rows_pad = _TILES * bpt
    grid = rows_pad // _IB
    et_v = edge_type.reshape(rows, 128).astype(jnp.int32)
    srco, srowo = pl.pallas_call(
        _make_idx_body(e),
        grid=(grid,),
        in_specs=[
            pl.BlockSpec((2, _IB * 128), lambda i: (0, i)),
            pl.BlockSpec((_IB, 128), lambda i: (i, 0)),
        ],
        out_specs=[
            pl.BlockSpec((_IB, 128), lambda i: (i, 0)),
            pl.BlockSpec((_IB, 128), lambda i: (i, 0)),
        ],
        out_shape=[
            jax.ShapeDtypeStruct((rows_pad, 128), jnp.int32),
            jax.ShapeDtypeStruct((rows_pad, 128), jnp.int32),
        ],
    )(edge_index, et_v)
    src3 = srco.reshape(_TILES, nch, _CH, _EB)
    srow3 = srowo.reshape(_TILES, nch, _CH, _EB)
    return src3, srow3


def _make_conv_body(relu):
    def body(h_ref, a_ref, c0_ref, c1_ref, w_ref, root_ref, bias_ref,
             out_ref):
        acc = lax.dot_general(h_ref[...], root_ref[...],
                              (((1,), (0,)), ((), ())),
                              preferred_element_type=jnp.float32)
        rc = 1.0 / jnp.maximum(c0_ref[...] + c1_ref[...], 1.0)
        rcx = jnp.concatenate(
            [jnp.broadcast_to(rc[:, r:r + 1], (_BN, _Q)) for r in range(_R)],
            axis=1)
        cat = jnp.concatenate([a_ref[q] * rcx for q in range(4)], axis=1)
        acc = acc + lax.dot_general(cat, w_ref[...],
                                    (((1,), (0,)), ((), ())),
                                    preferred_element_type=jnp.float32)
        acc = acc + bias_ref[...]
        if relu:
            acc = jnp.maximum(acc, 0.0)
        out_ref[...] = acc
    return body


def _conv_combine(h, a, cnt0, cnt1, w, root, bias, relu, out_rows):
    return pl.pallas_call(
        _make_conv_body(relu),
        grid=(_N // _BN,),
        in_specs=[
            pl.BlockSpec((_BN, _D), lambda i: (i, 0)),
            pl.BlockSpec((4, _BN, _D), lambda i: (0, i, 0)),
            pl.BlockSpec((_BN, _R), lambda i: (i, 0)),
            pl.BlockSpec((_BN, _R), lambda i: (i, 0)),
            pl.BlockSpec((4 * _D, _D), lambda i: (0, 0)),
            pl.BlockSpec((_D, _D), lambda i: (0, 0)),
            pl.BlockSpec((1, _D), lambda i: (0, 0)),
        ],
        out_specs=pl.BlockSpec((_BN, _D), lambda i: (i, 0)),
        out_shape=jax.ShapeDtypeStruct((out_rows, _D), jnp.float32),
    )(h, a, cnt0, cnt1, w, root, bias.reshape(1, _D))


_CH = 16
_NP = 10240


def _make_edge_body(nch, with_counts):
    def body(hs_ref, src_ref, srow_ref, agg_ref, *rest):
        if with_counts:
            cnt0_ref, cnt1_ref, rest = rest[0], rest[1], rest[2:]
        (agg_sh, cnt_sh, table_sh, zbuf, r0, r1, r2, r3, sidx, ridx, ones_v,
         rbuf, g0, g1, g2, g3, s0, s1, s2, s3, csem) = rest
        rows_bufs = (r0, r1, r2, r3)
        gsems = (g0, g1, g2, g3)
        ssems = (s0, s1, s2, s3)
        cid = lax.axis_index("c")
        sid = lax.axis_index("s")

        z16 = jnp.zeros((16,), jnp.float32)

        def _zero_zbuf(i, _):
            zbuf[i, pl.ds(0, 16)] = z16
            zbuf[i, pl.ds(16, 16)] = z16
            return 0
        lax.fori_loop(0, _ZR, _zero_zbuf, 0)

        def _zero_rbuf(i, _):
            rbuf[pl.ds(i * 16, 16)] = z16
            return 0
        lax.fori_loop(0, _STRIPE // 16, _zero_rbuf, 0)

        if with_counts:
            one16 = jnp.ones((16,), jnp.float32)
            for i in range(_EB // 16):
                ones_v[pl.ds(i * 16, 16)] = one16
            pltpu.sync_copy(rbuf, cnt_sh.at[pl.ds(sid * _STRIPE, _STRIPE)])

        ts0 = 632
        ts15 = _N - 15 * ts0
        half = nch // 2
        for p in range(2):
            q = cid + 2 * p

            @pl.when(sid != 15)
            def _():
                pltpu.async_copy(
                    hs_ref.at[pl.ds(sid * ts0, ts0), pl.ds(q * _Q, _Q)],
                    table_sh.at[pl.ds(sid * ts0, ts0), :], csem)

            @pl.when(sid == 15)
            def _():
                pltpu.async_copy(
                    hs_ref.at[pl.ds(15 * ts0, ts15), pl.ds(q * _Q, _Q)],
                    table_sh.at[pl.ds(15 * ts0, ts15), :], csem)

            nz = _STRIPE // _ZR
            for j in range(nz):
                k = j % _NBUF
                if j >= _NBUF:
                    pltpu.make_async_copy(
                        zbuf, agg_sh.at[pl.ds(0, _ZR), :], gsems[k]).wait()
                pltpu.async_copy(
                    zbuf, agg_sh.at[pl.ds(sid * _STRIPE + j * _ZR, _ZR), :],
                    gsems[k])
            for k in range(_NBUF):
                pltpu.make_async_copy(zbuf, agg_sh.at[pl.ds(0, _ZR), :],
                                      gsems[k]).wait()

            @pl.when(sid != 15)
            def _():
                pltpu.make_async_copy(
                    hs_ref.at[pl.ds(sid * ts0, ts0), pl.ds(q * _Q, _Q)],
                    table_sh.at[pl.ds(sid * ts0, ts0), :], csem).wait()

            @pl.when(sid == 15)
            def _():
                pltpu.make_async_copy(
                    hs_ref.at[pl.ds(15 * ts0, ts15), pl.ds(q * _Q, _Q)],
                    table_sh.at[pl.ds(15 * ts0, ts15), :], csem).wait()
            plsc.subcore_barrier()

            counting = with_counts and p == 0

            def _chunk(c, _):
                pltpu.sync_copy(src_ref.at[sid, c], sidx)
                pltpu.sync_copy(srow_ref.at[sid, c], ridx)
                for k in range(2):
                    pltpu.async_copy(table_sh.at[sidx.at[k]], rows_bufs[k],
                                     gsems[k])
                for j in range(_CH):
                    k = j % _NBUF
                    if j + 2 < _CH:
                        k2 = (j + 2) % _NBUF
                        if j - 2 >= 0:
                            pltpu.make_async_copy(
                                rows_bufs[k2], agg_sh.at[ridx.at[0]],
                                ssems[k2]).wait()
                        pltpu.async_copy(table_sh.at[sidx.at[j + 2]],
                                         rows_bufs[k2], gsems[k2])
                    pltpu.make_async_copy(table_sh.at[sidx.at[0]],
                                          rows_bufs[k], gsems[k]).wait()
                    pltpu.async_copy(rows_bufs[k], agg_sh.at[ridx.at[j]],
                                     ssems[k], add=True)
                    if counting:
                        @pl.when((cid == 0) == (c < half))
                        def _():
                            pltpu.async_copy(ones_v, cnt_sh.at[ridx.at[j]],
                                             csem, add=True)
                for k in range(_NBUF):
                    pltpu.make_async_copy(rows_bufs[k], agg_sh.at[ridx.at[0]],
                                          ssems[k]).wait()
                if counting:
                    @pl.when((cid == 0) == (c < half))
                    def _():
                        for j in range(_CH):
                            pltpu.make_async_copy(
                                ones_v, cnt_sh.at[ridx.at[0]], csem).wait()
                return 0
            lax.fori_loop(0, nch, _chunk, 0)
            plsc.subcore_barrier()

            pltpu.sync_copy(
                agg_sh.at[pl.ds(sid * _STRIPE, _STRIPE), :],
                agg_ref.at[pl.ds(q * _RN_PAD + sid * _STRIPE, _STRIPE), :])
            if with_counts and p == 0:
                @pl.when(cid == 0)
                def _():
                    pltpu.sync_copy(cnt_sh.at[pl.ds(sid * _STRIPE, _STRIPE)],
                                    cnt0_ref.at[pl.ds(sid * _STRIPE,
                                                      _STRIPE)])

                @pl.when(cid == 1)
                def _():
                    pltpu.sync_copy(cnt_sh.at[pl.ds(sid * _STRIPE, _STRIPE)],
                                    cnt1_ref.at[pl.ds(sid * _STRIPE,
                                                      _STRIPE)])
            plsc.subcore_barrier()

    return body


def _edge_pass(hs, src4, srow3, with_counts):
    nch = src4.shape[1]
    out_type = [jax.ShapeDtypeStruct((4 * _RN_PAD, _Q), jnp.float32)]
    if with_counts:
        out_type.append(jax.ShapeDtypeStruct((_RN_PAD,), jnp.float32))
        out_type.append(jax.ShapeDtypeStruct((_RN_PAD,), jnp.float32))
    scratch = (
        [
            pltpu.VMEM_SHARED((_RN_PAD, _Q), jnp.float32),
            pltpu.VMEM_SHARED((_RN_PAD,), jnp.float32),
            pltpu.VMEM_SHARED((_NP, _Q), jnp.float32),
            pltpu.VMEM((_ZR, _Q), jnp.float32),
        ]
        + [pltpu.VMEM((_EB, _Q), jnp.float32)] * _NBUF
        + [
            pltpu.VMEM((_CH, _EB), jnp.int32),
            pltpu.VMEM((_CH, _EB), jnp.int32),
            pltpu.VMEM((_EB,), jnp.float32),
            pltpu.VMEM((_STRIPE,), jnp.float32),
        ]
        + [pltpu.SemaphoreType.DMA] * (2 * _NBUF + 1)
    )
    mesh = plsc.VectorSubcoreMesh(core_axis_name="c", subcore_axis_name="s",
                                  num_cores=2, num_subcores=_TILES)
    fn = pl.kernel(
        _make_edge_body(nch, with_counts),
        out_type=tuple(out_type),
        mesh=mesh,
        scratch_types=scratch,
        compiler_params=pltpu.CompilerParams(use_tc_tiling_on_sc=False),
    )
    return fn(hs, src4, srow3)


def kernel(x, W_proj, b_proj, basis0, comp0, root0, bias0,
           basis1, comp1, root1, bias1, edge_index, edge_type):
    w0, w1 = _combine_weights(comp0, basis0, comp1, basis1)
    h = _project(x, W_proj, b_proj)
    src3, srow3 = _edge_indices(edge_index, edge_type)

    agg0, cnt0, cnt1 = _edge_pass(h, src3, srow3, with_counts=True)
    a0 = agg0.reshape(4, _RN_PAD // _R, _D)
    c0 = cnt0.reshape(_RN_PAD // _R, _R)
    c1 = cnt1.reshape(_RN_PAD // _R, _R)
    x1 = _conv_combine(h, a0, c0, c1, w0, root0, bias0,
                       relu=True, out_rows=_N)

    agg1 = _edge_pass(x1, src3, srow3, with_counts=False)[0]
    a1 = agg1.reshape(4, _RN_PAD // _R, _D)
    out = _conv_combine(x1, a1, c0, c1, w1, root1, bias1,
                        relu=False, out_rows=_N)
    return out, h, h


# speedup vs baseline: 21.5687x; 1.0289x over previous
import jax
import jax.numpy as jnp
from jax import lax
from jax.experimental import pallas as pl
from jax.experimental.pallas import tpu as pltpu
from jax.experimental.pallas import tpu_sc as plsc

_N = 10000
_D = 128
_R = 4
_NBASES = 8
_Q = 32
_RN = _N * _R
_RN_PAD = 40960
_EB = 128
_TILES = 16
_NBUF = 4
_ZR = 64
_STRIPE = _RN_PAD // _TILES


def _weights_body(comp0_ref, comp1_ref, basis0_ref, basis1_ref, w0_ref, w1_ref):
    for c_ref, b_ref, w_ref in ((comp0_ref, basis0_ref, w0_ref),
                                (comp1_ref, basis1_ref, w1_ref)):
        b = b_ref[...]
        for r in range(_R):
            acc = c_ref[r, 0] * b[0]
            for k in range(1, _NBASES):
                acc = acc + c_ref[r, k] * b[k]
            for q in range(4):
                w_ref[pl.ds(q * _D + r * _Q, _Q), :] = \
                    acc[q * _Q:(q + 1) * _Q, :]


def _combine_weights(comp0, basis0, comp1, basis1):
    return pl.pallas_call(
        _weights_body,
        in_specs=[
            pl.BlockSpec(memory_space=pltpu.SMEM),
            pl.BlockSpec(memory_space=pltpu.SMEM),
            pl.BlockSpec((_NBASES, _D, _D), lambda: (0, 0, 0)),
            pl.BlockSpec((_NBASES, _D, _D), lambda: (0, 0, 0)),
        ],
        out_specs=[
            pl.BlockSpec((4 * _D, _D), lambda: (0, 0)),
            pl.BlockSpec((4 * _D, _D), lambda: (0, 0)),
        ],
        out_shape=[
            jax.ShapeDtypeStruct((4 * _D, _D), jnp.float32),
            jax.ShapeDtypeStruct((4 * _D, _D), jnp.float32),
        ],
    )(comp0, comp1, basis0, basis1)


_BN = 1000


def _proj_body(x_ref, wt_ref, b_ref, h_ref, hb_ref):
    h = lax.dot_general(x_ref[...], wt_ref[...], (((1,), (0,)), ((), ())),
                        preferred_element_type=jnp.float32)
    h = jnp.maximum(h + b_ref[...], 0.0)
    h_ref[...] = h
    hb_ref[...] = h.astype(jnp.bfloat16)


def _project(x, W_proj, b_proj):
    return pl.pallas_call(
        _proj_body,
        grid=(_N // _BN,),
        in_specs=[
            pl.BlockSpec((_BN, _D), lambda i: (i, 0)),
            pl.BlockSpec((_D, _D), lambda i: (0, 0)),
            pl.BlockSpec((1, _D), lambda i: (0, 0)),
        ],
        out_specs=[
            pl.BlockSpec((_BN, _D), lambda i: (i, 0)),
            pl.BlockSpec((_BN, _D), lambda i: (i, 0)),
        ],
        out_shape=[
            jax.ShapeDtypeStruct((_N, _D), jnp.float32),
            jax.ShapeDtypeStruct((_N, _D), jnp.bfloat16),
        ],
    )(x, W_proj.T, b_proj.reshape(1, _D))


_IB = 128


def _make_idx_body(e):
    def body(ei_ref, et_ref, srco_ref, srowo_ref):
        i = pl.program_id(0)
        g = (i * (_IB * 128)
             + lax.broadcasted_iota(jnp.int32, (_IB, 128), 0) * 128
             + lax.broadcasted_iota(jnp.int32, (_IB, 128), 1))
        mask = g < e
        src = ei_ref[0].reshape(_IB, 128)
        dst = ei_ref[1].reshape(_IB, 128)
        srco_ref[...] = jnp.where(mask, src, g % _N)
        srowo_ref[...] = jnp.where(mask, dst * _R + et_ref[...],
                                   _RN + g % (_RN_PAD - _RN))
    return body


def _edge_indices(edge_index, edge_type):
    e = edge_index.shape[1]
    assert e % 128 == 0
    rows = e // 128
    bpt = -(-e // (_TILES * _EB))
    if bpt % _CH:
        bpt += _CH - bpt % _CH
    nch = bpt // _CH
    rows_pad = _TILES * bpt
    grid = rows_pad // _IB
    et_v = edge_type.reshape(rows, 128).astype(jnp.int32)
    srco, srowo = pl.pallas_call(
        _make_idx_body(e),
        grid=(grid,),
        in_specs=[
            pl.BlockSpec((2, _IB * 128), lambda i: (0, i)),
            pl.BlockSpec((_IB, 128), lambda i: (i, 0)),
        ],
        out_specs=[
            pl.BlockSpec((_IB, 128), lambda i: (i, 0)),
            pl.BlockSpec((_IB, 128), lambda i: (i, 0)),
        ],
        out_shape=[
            jax.ShapeDtypeStruct((rows_pad, 128), jnp.int32),
            jax.ShapeDtypeStruct((rows_pad, 128), jnp.int32),
        ],
    )(edge_index, et_v)
    src3 = srco.reshape(_TILES, nch, _CH, _EB)
    srow3 = srowo.reshape(_TILES, nch, _CH, _EB)
    return src3, srow3


def _make_conv_body(relu, bf_out):
    def body(h_ref, a_ref, c0_ref, c1_ref, w_ref, root_ref, bias_ref,
             *out_refs):
        acc = lax.dot_general(h_ref[...], root_ref[...],
                              (((1,), (0,)), ((), ())),
                              preferred_element_type=jnp.float32)
        rc = 1.0 / jnp.maximum(c0_ref[...] + c1_ref[...], 1.0)
        rcx = jnp.concatenate(
            [jnp.broadcast_to(rc[:, r:r + 1], (_BN, _Q)) for r in range(_R)],
            axis=1)
        cat = jnp.concatenate(
            [a_ref[q].astype(jnp.float32) * rcx for q in range(4)], axis=1)
        acc = acc + lax.dot_general(cat, w_ref[...],
                                    (((1,), (0,)), ((), ())),
                                    preferred_element_type=jnp.float32)
        acc = acc + bias_ref[...]
        if relu:
            acc = jnp.maximum(acc, 0.0)
        out_refs[0][...] = acc
        if bf_out:
            out_refs[1][...] = acc.astype(jnp.bfloat16)
    return body


def _conv_combine(h, a, cnt0, cnt1, w, root, bias, relu, bf_out):
    out_specs = [pl.BlockSpec((_BN, _D), lambda i: (i, 0))]
    out_shape = [jax.ShapeDtypeStruct((_N, _D), jnp.float32)]
    if bf_out:
        out_specs.append(pl.BlockSpec((_BN, _D), lambda i: (i, 0)))
        out_shape.append(jax.ShapeDtypeStruct((_N, _D), jnp.bfloat16))
    return pl.pallas_call(
        _make_conv_body(relu, bf_out),
        grid=(_N // _BN,),
        in_specs=[
            pl.BlockSpec((_BN, _D), lambda i: (i, 0)),
            pl.BlockSpec((4, _BN, _D), lambda i: (0, i, 0)),
            pl.BlockSpec((_BN, _R), lambda i: (i, 0)),
            pl.BlockSpec((_BN, _R), lambda i: (i, 0)),
            pl.BlockSpec((4 * _D, _D), lambda i: (0, 0)),
            pl.BlockSpec((_D, _D), lambda i: (0, 0)),
            pl.BlockSpec((1, _D), lambda i: (0, 0)),
        ],
        out_specs=out_specs,
        out_shape=out_shape,
    )(h, a, cnt0, cnt1, w, root, bias.reshape(1, _D))


_CH = 16
_NP = 10240


def _make_edge_body(nch, with_counts):
    def body(hs_ref, src_ref, srow_ref, agg_ref, *rest):
        if with_counts:
            cnt0_ref, cnt1_ref, rest = rest[0], rest[1], rest[2:]
        (agg_sh, cnt_sh, table_sh, zbuf, r0, r1, r2, r3, sidx, ridx, ones_v,
         rbuf, g0, g1, g2, g3, s0, s1, s2, s3, csem) = rest
        rows_bufs = (r0, r1, r2, r3)
        gsems = (g0, g1, g2, g3)
        ssems = (s0, s1, s2, s3)
        cid = lax.axis_index("c")
        sid = lax.axis_index("s")

        z16 = jnp.zeros((16,), jnp.float32)
        z32 = jnp.zeros((32,), jnp.bfloat16)

        def _zero_zbuf(i, _):
            zbuf[i, :] = z32
            return 0
        lax.fori_loop(0, _ZR, _zero_zbuf, 0)

        def _zero_rbuf(i, _):
            rbuf[pl.ds(i * 16, 16)] = z16
            return 0
        lax.fori_loop(0, _STRIPE // 16, _zero_rbuf, 0)

        if with_counts:
            one16 = jnp.ones((16,), jnp.float32)
            for i in range(_EB // 16):
                ones_v[pl.ds(i * 16, 16)] = one16
            pltpu.sync_copy(rbuf, cnt_sh.at[pl.ds(sid * _STRIPE, _STRIPE)])

        ts0 = 632
        ts15 = _N - 15 * ts0
        half = nch // 2
        for p in range(2):
            q = cid + 2 * p

            @pl.when(sid != 15)
            def _():
                pltpu.async_copy(
                    hs_ref.at[pl.ds(sid * ts0, ts0), pl.ds(q * _Q, _Q)],
                    table_sh.at[pl.ds(sid * ts0, ts0), :], csem)

            @pl.when(sid == 15)
            def _():
                pltpu.async_copy(
                    hs_ref.at[pl.ds(15 * ts0, ts15), pl.ds(q * _Q, _Q)],
                    table_sh.at[pl.ds(15 * ts0, ts15), :], csem)

            nz = _STRIPE // _ZR
            for j in range(nz):
                k = j % _NBUF
                if j >= _NBUF:
                    pltpu.make_async_copy(
                        zbuf, agg_sh.at[pl.ds(0, _ZR), :], gsems[k]).wait()
                pltpu.async_copy(
                    zbuf, agg_sh.at[pl.ds(sid * _STRIPE + j * _ZR, _ZR), :],
                    gsems[k])
            for k in range(_NBUF):
                pltpu.make_async_copy(zbuf, agg_sh.at[pl.ds(0, _ZR), :],
                                      gsems[k]).wait()

            @pl.when(sid != 15)
            def _():
                pltpu.make_async_copy(
                    hs_ref.at[pl.ds(sid * ts0, ts0), pl.ds(q * _Q, _Q)],
                    table_sh.at[pl.ds(sid * ts0, ts0), :], csem).wait()

            @pl.when(sid == 15)
            def _():
                pltpu.make_async_copy(
                    hs_ref.at[pl.ds(15 * ts0, ts15), pl.ds(q * _Q, _Q)],
                    table_sh.at[pl.ds(15 * ts0, ts15), :], csem).wait()
            plsc.subcore_barrier()

            counting = with_counts and p == 0

            def _chunk(c, _):
                pltpu.sync_copy(src_ref.at[sid, c], sidx)
                pltpu.sync_copy(srow_ref.at[sid, c], ridx)
                for k in range(2):
                    pltpu.async_copy(table_sh.at[sidx.at[k]], rows_bufs[k],
                                     gsems[k])
                for j in range(_CH):
                    k = j % _NBUF
                    if j + 2 < _CH:
                        k2 = (j + 2) % _NBUF
                        if j - 2 >= 0:
                            pltpu.make_async_copy(
                                rows_bufs[k2], agg_sh.at[ridx.at[0]],
                                ssems[k2]).wait()
                        pltpu.async_copy(table_sh.at[sidx.at[j + 2]],
                                         rows_bufs[k2], gsems[k2])
                    pltpu.make_async_copy(table_sh.at[sidx.at[0]],
                                          rows_bufs[k], gsems[k]).wait()
                    pltpu.async_copy(rows_bufs[k], agg_sh.at[ridx.at[j]],
                                     ssems[k], add=True)
                    if counting:
                        @pl.when((cid == 0) == (c < half))
                        def _():
                            pltpu.async_copy(ones_v, cnt_sh.at[ridx.at[j]],
                                             csem, add=True)
                for k in range(_NBUF):
                    pltpu.make_async_copy(rows_bufs[k], agg_sh.at[ridx.at[0]],
                                          ssems[k]).wait()
                if counting:
                    @pl.when((cid == 0) == (c < half))
                    def _():
                        for j in range(_CH):
                            pltpu.make_async_copy(
                                ones_v, cnt_sh.at[ridx.at[0]], csem).wait()
                return 0
            lax.fori_loop(0, nch, _chunk, 0)
            plsc.subcore_barrier()

            pltpu.sync_copy(
                agg_sh.at[pl.ds(sid * _STRIPE, _STRIPE), :],
                agg_ref.at[pl.ds(q * _RN_PAD + sid * _STRIPE, _STRIPE), :])
            if with_counts and p == 0:
                @pl.when(cid == 0)
                def _():
                    pltpu.sync_copy(cnt_sh.at[pl.ds(sid * _STRIPE, _STRIPE)],
                                    cnt0_ref.at[pl.ds(sid * _STRIPE,
                                                      _STRIPE)])

                @pl.when(cid == 1)
                def _():
                    pltpu.sync_copy(cnt_sh.at[pl.ds(sid * _STRIPE, _STRIPE)],
                                    cnt1_ref.at[pl.ds(sid * _STRIPE,
                                                      _STRIPE)])
            plsc.subcore_barrier()

    return body


def _edge_pass(hs, src4, srow3, with_counts):
    nch = src4.shape[1]
    out_type = [jax.ShapeDtypeStruct((4 * _RN_PAD, _Q), jnp.bfloat16)]
    if with_counts:
        out_type.append(jax.ShapeDtypeStruct((_RN_PAD,), jnp.float32))
        out_type.append(jax.ShapeDtypeStruct((_RN_PAD,), jnp.float32))
    scratch = (
        [
            pltpu.VMEM_SHARED((_RN_PAD, _Q), jnp.bfloat16),
            pltpu.VMEM_SHARED((_RN_PAD,), jnp.float32),
            pltpu.VMEM_SHARED((_NP, _Q), jnp.bfloat16),
            pltpu.VMEM((_ZR, _Q), jnp.bfloat16),
        ]
        + [pltpu.VMEM((_EB, _Q), jnp.bfloat16)] * _NBUF
        + [
            pltpu.VMEM((_CH, _EB), jnp.int32),
            pltpu.VMEM((_CH, _EB), jnp.int32),
            pltpu.VMEM((_EB,), jnp.float32),
            pltpu.VMEM((_STRIPE,), jnp.float32),
        ]
        + [pltpu.SemaphoreType.DMA] * (2 * _NBUF + 1)
    )
    mesh = plsc.VectorSubcoreMesh(core_axis_name="c", subcore_axis_name="s",
                                  num_cores=2, num_subcores=_TILES)
    fn = pl.kernel(
        _make_edge_body(nch, with_counts),
        out_type=tuple(out_type),
        mesh=mesh,
        scratch_types=scratch,
        compiler_params=pltpu.CompilerParams(use_tc_tiling_on_sc=False),
    )
    return fn(hs, src4, srow3)


def kernel(x, W_proj, b_proj, basis0, comp0, root0, bias0,
           basis1, comp1, root1, bias1, edge_index, edge_type):
    w0, w1 = _combine_weights(comp0, basis0, comp1, basis1)
    h, hb = _project(x, W_proj, b_proj)
    src3, srow3 = _edge_indices(edge_index, edge_type)

    agg0, cnt0, cnt1 = _edge_pass(hb, src3, srow3, with_counts=True)
    a0 = agg0.reshape(4, _RN_PAD // _R, _D)
    c0 = cnt0.reshape(_RN_PAD // _R, _R)
    c1 = cnt1.reshape(_RN_PAD // _R, _R)
    x1, x1b = _conv_combine(h, a0, c0, c1, w0, root0, bias0,
                            relu=True, bf_out=True)

    agg1 = _edge_pass(x1b, src3, srow3, with_counts=False)[0]
    a1 = agg1.reshape(4, _RN_PAD // _R, _D)
    out = _conv_combine(x1, a1, c0, c1, w1, root1, bias1,
                        relu=False, bf_out=False)[0]
    return out, h, h


# speedup vs baseline: 22.7132x; 1.0531x over previous
import jax
import jax.numpy as jnp
from jax import lax
from jax.experimental import pallas as pl
from jax.experimental.pallas import tpu as pltpu
from jax.experimental.pallas import tpu_sc as plsc

_N = 10000
_D = 128
_R = 4
_NBASES = 8
_Q = 32
_RN = _N * _R
_RN_PAD = 40960
_EB = 128
_TILES = 16
_NBUF = 8
_PF = 4
_ZR = 64
_STRIPE = _RN_PAD // _TILES


def _weights_body(comp0_ref, comp1_ref, basis0_ref, basis1_ref, w0_ref, w1_ref):
    for c_ref, b_ref, w_ref in ((comp0_ref, basis0_ref, w0_ref),
                                (comp1_ref, basis1_ref, w1_ref)):
        b = b_ref[...]
        for r in range(_R):
            acc = c_ref[r, 0] * b[0]
            for k in range(1, _NBASES):
                acc = acc + c_ref[r, k] * b[k]
            for q in range(4):
                w_ref[pl.ds(q * _D + r * _Q, _Q), :] = \
                    acc[q * _Q:(q + 1) * _Q, :]


def _combine_weights(comp0, basis0, comp1, basis1):
    return pl.pallas_call(
        _weights_body,
        in_specs=[
            pl.BlockSpec(memory_space=pltpu.SMEM),
            pl.BlockSpec(memory_space=pltpu.SMEM),
            pl.BlockSpec((_NBASES, _D, _D), lambda: (0, 0, 0)),
            pl.BlockSpec((_NBASES, _D, _D), lambda: (0, 0, 0)),
        ],
        out_specs=[
            pl.BlockSpec((4 * _D, _D), lambda: (0, 0)),
            pl.BlockSpec((4 * _D, _D), lambda: (0, 0)),
        ],
        out_shape=[
            jax.ShapeDtypeStruct((4 * _D, _D), jnp.float32),
            jax.ShapeDtypeStruct((4 * _D, _D), jnp.float32),
        ],
    )(comp0, comp1, basis0, basis1)


_BN = 1000


def _proj_body(x_ref, wt_ref, b_ref, h_ref, hb_ref):
    h = lax.dot_general(x_ref[...], wt_ref[...], (((1,), (0,)), ((), ())),
                        preferred_element_type=jnp.float32)
    h = jnp.maximum(h + b_ref[...], 0.0)
    h_ref[...] = h
    hb_ref[...] = h.astype(jnp.bfloat16)


def _project(x, W_proj, b_proj):
    return pl.pallas_call(
        _proj_body,
        grid=(_N // _BN,),
        in_specs=[
            pl.BlockSpec((_BN, _D), lambda i: (i, 0)),
            pl.BlockSpec((_D, _D), lambda i: (0, 0)),
            pl.BlockSpec((1, _D), lambda i: (0, 0)),
        ],
        out_specs=[
            pl.BlockSpec((_BN, _D), lambda i: (i, 0)),
            pl.BlockSpec((_BN, _D), lambda i: (i, 0)),
        ],
        out_shape=[
            jax.ShapeDtypeStruct((_N, _D), jnp.float32),
            jax.ShapeDtypeStruct((_N, _D), jnp.bfloat16),
        ],
    )(x, W_proj.T, b_proj.reshape(1, _D))


_IB = 128


def _make_idx_body(e):
    def body(ei_ref, et_ref, srco_ref, srowo_ref):
        i = pl.program_id(0)
        g = (i * (_IB * 128)
             + lax.broadcasted_iota(jnp.int32, (_IB, 128), 0) * 128
             + lax.broadcasted_iota(jnp.int32, (_IB, 128), 1))
        mask = g < e
        src = ei_ref[0].reshape(_IB, 128)
        dst = ei_ref[1].reshape(_IB, 128)
        srco_ref[...] = jnp.where(mask, src, g % _N)
        srowo_ref[...] = jnp.where(mask, dst * _R + et_ref[...],
                                   _RN + g % (_RN_PAD - _RN))
    return body


def _edge_indices(edge_index, edge_type):
    e = edge_index.shape[1]
    assert e % 128 == 0
    rows = e // 128
    bpt = -(-e // (_TILES * _EB))
    if bpt % _CH:
        bpt += _CH - bpt % _CH
    nch = bpt // _CH
    rows_pad = _TILES * bpt
    grid = rows_pad // _IB
    et_v = edge_type.reshape(rows, 128).astype(jnp.int32)
    srco, srowo = pl.pallas_call(
        _make_idx_body(e),
        grid=(grid,),
        in_specs=[
            pl.BlockSpec((2, _IB * 128), lambda i: (0, i)),
            pl.BlockSpec((_IB, 128), lambda i: (i, 0)),
        ],
        out_specs=[
            pl.BlockSpec((_IB, 128), lambda i: (i, 0)),
            pl.BlockSpec((_IB, 128), lambda i: (i, 0)),
        ],
        out_shape=[
            jax.ShapeDtypeStruct((rows_pad, 128), jnp.int32),
            jax.ShapeDtypeStruct((rows_pad, 128), jnp.int32),
        ],
    )(edge_index, et_v)
    src3 = srco.reshape(_TILES, nch, _CH, _EB)
    srow3 = srowo.reshape(_TILES, nch, _CH, _EB)
    return src3, srow3


def _make_conv_body(relu, bf_out):
    def body(h_ref, a_ref, c0_ref, c1_ref, w_ref, root_ref, bias_ref,
             *out_refs):
        acc = lax.dot_general(h_ref[...], root_ref[...],
                              (((1,), (0,)), ((), ())),
                              preferred_element_type=jnp.float32)
        rc = 1.0 / jnp.maximum(c0_ref[...] + c1_ref[...], 1.0)
        rcx = jnp.concatenate(
            [jnp.broadcast_to(rc[:, r:r + 1], (_BN, _Q)) for r in range(_R)],
            axis=1)
        cat = jnp.concatenate(
            [a_ref[q].astype(jnp.float32) * rcx for q in range(4)], axis=1)
        acc = acc + lax.dot_general(cat, w_ref[...],
                                    (((1,), (0,)), ((), ())),
                                    preferred_element_type=jnp.float32)
        acc = acc + bias_ref[...]
        if relu:
            acc = jnp.maximum(acc, 0.0)
        out_refs[0][...] = acc
        if bf_out:
            out_refs[1][...] = acc.astype(jnp.bfloat16)
    return body


def _conv_combine(h, a, cnt0, cnt1, w, root, bias, relu, bf_out):
    out_specs = [pl.BlockSpec((_BN, _D), lambda i: (i, 0))]
    out_shape = [jax.ShapeDtypeStruct((_N, _D), jnp.float32)]
    if bf_out:
        out_specs.append(pl.BlockSpec((_BN, _D), lambda i: (i, 0)))
        out_shape.append(jax.ShapeDtypeStruct((_N, _D), jnp.bfloat16))
    return pl.pallas_call(
        _make_conv_body(relu, bf_out),
        grid=(_N // _BN,),
        in_specs=[
            pl.BlockSpec((_BN, _D), lambda i: (i, 0)),
            pl.BlockSpec((4, _BN, _D), lambda i: (0, i, 0)),
            pl.BlockSpec((_BN, _R), lambda i: (i, 0)),
            pl.BlockSpec((_BN, _R), lambda i: (i, 0)),
            pl.BlockSpec((4 * _D, _D), lambda i: (0, 0)),
            pl.BlockSpec((_D, _D), lambda i: (0, 0)),
            pl.BlockSpec((1, _D), lambda i: (0, 0)),
        ],
        out_specs=out_specs,
        out_shape=out_shape,
    )(h, a, cnt0, cnt1, w, root, bias.reshape(1, _D))


_CH = 32
_NP = 10240


def _make_edge_body(nch, with_counts):
    def body(hs_ref, src_ref, srow_ref, agg_ref, *rest):
        if with_counts:
            cnt0_ref, cnt1_ref, rest = rest[0], rest[1], rest[2:]
        agg_sh, cnt_sh, table_sh, zbuf = rest[:4]
        rows_bufs = rest[4:4 + _NBUF]
        sidx, ridx, ones_v, rbuf = rest[4 + _NBUF:8 + _NBUF]
        gsems = rest[8 + _NBUF:8 + 2 * _NBUF]
        ssems = rest[8 + 2 * _NBUF:8 + 3 * _NBUF]
        csem = rest[8 + 3 * _NBUF]
        cid = lax.axis_index("c")
        sid = lax.axis_index("s")

        z16 = jnp.zeros((16,), jnp.float32)
        z32 = jnp.zeros((32,), jnp.bfloat16)

        def _zero_zbuf(i, _):
            zbuf[i, :] = z32
            return 0
        lax.fori_loop(0, _ZR, _zero_zbuf, 0)

        def _zero_rbuf(i, _):
            rbuf[pl.ds(i * 16, 16)] = z16
            return 0
        lax.fori_loop(0, _STRIPE // 16, _zero_rbuf, 0)

        if with_counts:
            one16 = jnp.ones((16,), jnp.float32)
            for i in range(_EB // 16):
                ones_v[pl.ds(i * 16, 16)] = one16
            pltpu.sync_copy(rbuf, cnt_sh.at[pl.ds(sid * _STRIPE, _STRIPE)])

        ts0 = 632
        ts15 = _N - 15 * ts0
        half = nch // 2
        for p in range(2):
            q = cid + 2 * p

            @pl.when(sid != 15)
            def _():
                pltpu.async_copy(
                    hs_ref.at[pl.ds(sid * ts0, ts0), pl.ds(q * _Q, _Q)],
                    table_sh.at[pl.ds(sid * ts0, ts0), :], csem)

            @pl.when(sid == 15)
            def _():
                pltpu.async_copy(
                    hs_ref.at[pl.ds(15 * ts0, ts15), pl.ds(q * _Q, _Q)],
                    table_sh.at[pl.ds(15 * ts0, ts15), :], csem)

            nz = _STRIPE // _ZR
            for j in range(nz):
                k = j % _NBUF
                if j >= _NBUF:
                    pltpu.make_async_copy(
                        zbuf, agg_sh.at[pl.ds(0, _ZR), :], gsems[k]).wait()
                pltpu.async_copy(
                    zbuf, agg_sh.at[pl.ds(sid * _STRIPE + j * _ZR, _ZR), :],
                    gsems[k])
            for k in range(_NBUF):
                pltpu.make_async_copy(zbuf, agg_sh.at[pl.ds(0, _ZR), :],
                                      gsems[k]).wait()

            @pl.when(sid != 15)
            def _():
                pltpu.make_async_copy(
                    hs_ref.at[pl.ds(sid * ts0, ts0), pl.ds(q * _Q, _Q)],
                    table_sh.at[pl.ds(sid * ts0, ts0), :], csem).wait()

            @pl.when(sid == 15)
            def _():
                pltpu.make_async_copy(
                    hs_ref.at[pl.ds(15 * ts0, ts15), pl.ds(q * _Q, _Q)],
                    table_sh.at[pl.ds(15 * ts0, ts15), :], csem).wait()
            plsc.subcore_barrier()

            counting = with_counts and p == 0

            def _chunk(c, _):
                pltpu.sync_copy(src_ref.at[sid, c], sidx)
                pltpu.sync_copy(srow_ref.at[sid, c], ridx)
                for k in range(_PF):
                    pltpu.async_copy(table_sh.at[sidx.at[k]], rows_bufs[k],
                                     gsems[k])
                for j in range(_CH):
                    k = j % _NBUF
                    if j + _PF < _CH:
                        k2 = (j + _PF) % _NBUF
                        if j + _PF - _NBUF >= 0:
                            pltpu.make_async_copy(
                                rows_bufs[k2], agg_sh.at[ridx.at[0]],
                                ssems[k2]).wait()
                        pltpu.async_copy(table_sh.at[sidx.at[j + _PF]],
                                         rows_bufs[k2], gsems[k2])
                    pltpu.make_async_copy(table_sh.at[sidx.at[0]],
                                          rows_bufs[k], gsems[k]).wait()
                    pltpu.async_copy(rows_bufs[k], agg_sh.at[ridx.at[j]],
                                     ssems[k], add=True)
                    if counting:
                        @pl.when((cid == 0) == (c < half))
                        def _():
                            pltpu.async_copy(ones_v, cnt_sh.at[ridx.at[j]],
                                             csem, add=True)
                for j in range(max(_CH - _NBUF, 0), _CH):
                    k = j % _NBUF
                    pltpu.make_async_copy(rows_bufs[k], agg_sh.at[ridx.at[0]],
                                          ssems[k]).wait()
                if counting:
                    @pl.when((cid == 0) == (c < half))
                    def _():
                        for j in range(_CH):
                            pltpu.make_async_copy(
                                ones_v, cnt_sh.at[ridx.at[0]], csem).wait()
                return 0
            lax.fori_loop(0, nch, _chunk, 0)
            plsc.subcore_barrier()

            pltpu.sync_copy(
                agg_sh.at[pl.ds(sid * _STRIPE, _STRIPE), :],
                agg_ref.at[pl.ds(q * _RN_PAD + sid * _STRIPE, _STRIPE), :])
            if with_counts and p == 0:
                @pl.when(cid == 0)
                def _():
                    pltpu.sync_copy(cnt_sh.at[pl.ds(sid * _STRIPE, _STRIPE)],
                                    cnt0_ref.at[pl.ds(sid * _STRIPE,
                                                      _STRIPE)])

                @pl.when(cid == 1)
                def _():
                    pltpu.sync_copy(cnt_sh.at[pl.ds(sid * _STRIPE, _STRIPE)],
                                    cnt1_ref.at[pl.ds(sid * _STRIPE,
                                                      _STRIPE)])
            plsc.subcore_barrier()

    return body


def _edge_pass(hs, src4, srow3, with_counts):
    nch = src4.shape[1]
    out_type = [jax.ShapeDtypeStruct((4 * _RN_PAD, _Q), jnp.bfloat16)]
    if with_counts:
        out_type.append(jax.ShapeDtypeStruct((_RN_PAD,), jnp.float32))
        out_type.append(jax.ShapeDtypeStruct((_RN_PAD,), jnp.float32))
    scratch = (
        [
            pltpu.VMEM_SHARED((_RN_PAD, _Q), jnp.bfloat16),
            pltpu.VMEM_SHARED((_RN_PAD,), jnp.float32),
            pltpu.VMEM_SHARED((_NP, _Q), jnp.bfloat16),
            pltpu.VMEM((_ZR, _Q), jnp.bfloat16),
        ]
        + [pltpu.VMEM((_EB, _Q), jnp.bfloat16)] * _NBUF
        + [
            pltpu.VMEM((_CH, _EB), jnp.int32),
            pltpu.VMEM((_CH, _EB), jnp.int32),
            pltpu.VMEM((_EB,), jnp.float32),
            pltpu.VMEM((_STRIPE,), jnp.float32),
        ]
        + [pltpu.SemaphoreType.DMA] * (2 * _NBUF + 1)
    )
    mesh = plsc.VectorSubcoreMesh(core_axis_name="c", subcore_axis_name="s",
                                  num_cores=2, num_subcores=_TILES)
    fn = pl.kernel(
        _make_edge_body(nch, with_counts),
        out_type=tuple(out_type),
        mesh=mesh,
        scratch_types=scratch,
        compiler_params=pltpu.CompilerParams(use_tc_tiling_on_sc=False),
    )
    return fn(hs, src4, srow3)


def kernel(x, W_proj, b_proj, basis0, comp0, root0, bias0,
           basis1, comp1, root1, bias1, edge_index, edge_type):
    w0, w1 = _combine_weights(comp0, basis0, comp1, basis1)
    h, hb = _project(x, W_proj, b_proj)
    src3, srow3 = _edge_indices(edge_index, edge_type)

    agg0, cnt0, cnt1 = _edge_pass(hb, src3, srow3, with_counts=True)
    a0 = agg0.reshape(4, _RN_PAD // _R, _D)
    c0 = cnt0.reshape(_RN_PAD // _R, _R)
    c1 = cnt1.reshape(_RN_PAD // _R, _R)
    x1, x1b = _conv_combine(h, a0, c0, c1, w0, root0, bias0,
                            relu=True, bf_out=True)

    agg1 = _edge_pass(x1b, src3, srow3, with_counts=False)[0]
    a1 = agg1.reshape(4, _RN_PAD // _R, _D)
    out = _conv_combine(x1, a1, c0, c1, w1, root1, bias1,
                        relu=False, bf_out=False)[0]
    return out, h, h


# speedup vs baseline: 25.6598x; 1.1297x over previous
import jax
import jax.numpy as jnp
from jax import lax
from jax.experimental import pallas as pl
from jax.experimental.pallas import tpu as pltpu
from jax.experimental.pallas import tpu_sc as plsc

_N = 10000
_D = 128
_R = 4
_NBASES = 8
_Q = 32
_RN = _N * _R
_RN_PAD = 40960
_EB = 128
_TILES = 16
_NBUF = 8
_PF = 4
_ZR = 64
_STRIPE = _RN_PAD // _TILES


def _weights_body(comp0_ref, comp1_ref, basis0_ref, basis1_ref, w0_ref, w1_ref):
    for c_ref, b_ref, w_ref in ((comp0_ref, basis0_ref, w0_ref),
                                (comp1_ref, basis1_ref, w1_ref)):
        b = b_ref[...]
        for r in range(_R):
            acc = c_ref[r, 0] * b[0]
            for k in range(1, _NBASES):
                acc = acc + c_ref[r, k] * b[k]
            for q in range(4):
                w_ref[pl.ds(q * _D + r * _Q, _Q), :] = \
                    acc[q * _Q:(q + 1) * _Q, :]


def _combine_weights(comp0, basis0, comp1, basis1):
    return pl.pallas_call(
        _weights_body,
        in_specs=[
            pl.BlockSpec(memory_space=pltpu.SMEM),
            pl.BlockSpec(memory_space=pltpu.SMEM),
            pl.BlockSpec((_NBASES, _D, _D), lambda: (0, 0, 0)),
            pl.BlockSpec((_NBASES, _D, _D), lambda: (0, 0, 0)),
        ],
        out_specs=[
            pl.BlockSpec((4 * _D, _D), lambda: (0, 0)),
            pl.BlockSpec((4 * _D, _D), lambda: (0, 0)),
        ],
        out_shape=[
            jax.ShapeDtypeStruct((4 * _D, _D), jnp.float32),
            jax.ShapeDtypeStruct((4 * _D, _D), jnp.float32),
        ],
    )(comp0, comp1, basis0, basis1)


_BN = 1000


def _proj_body(x_ref, wt_ref, b_ref, h_ref, hb_ref):
    h = lax.dot_general(x_ref[...], wt_ref[...], (((1,), (0,)), ((), ())),
                        preferred_element_type=jnp.float32)
    h = jnp.maximum(h + b_ref[...], 0.0)
    h_ref[...] = h
    hb_ref[...] = h.astype(jnp.bfloat16)


def _project(x, W_proj, b_proj):
    return pl.pallas_call(
        _proj_body,
        grid=(_N // _BN,),
        in_specs=[
            pl.BlockSpec((_BN, _D), lambda i: (i, 0)),
            pl.BlockSpec((_D, _D), lambda i: (0, 0)),
            pl.BlockSpec((1, _D), lambda i: (0, 0)),
        ],
        out_specs=[
            pl.BlockSpec((_BN, _D), lambda i: (i, 0)),
            pl.BlockSpec((_BN, _D), lambda i: (i, 0)),
        ],
        out_shape=[
            jax.ShapeDtypeStruct((_N, _D), jnp.float32),
            jax.ShapeDtypeStruct((_N, _D), jnp.bfloat16),
        ],
    )(x, W_proj.T, b_proj.reshape(1, _D))


_IB = 128


def _make_idx_body(e):
    def body(ei_ref, et_ref, srco_ref, srowo_ref):
        i = pl.program_id(0)
        g = (i * (_IB * 128)
             + lax.broadcasted_iota(jnp.int32, (_IB, 128), 0) * 128
             + lax.broadcasted_iota(jnp.int32, (_IB, 128), 1))
        mask = g < e
        src = ei_ref[0].reshape(_IB, 128)
        dst = ei_ref[1].reshape(_IB, 128)
        srco_ref[...] = jnp.where(mask, src, g % _N)
        srowo_ref[...] = jnp.where(mask, dst * _R + et_ref[...],
                                   _RN + g % (_RN_PAD - _RN))
    return body


def _edge_indices(edge_index, edge_type):
    e = edge_index.shape[1]
    assert e % 128 == 0
    rows = e // 128
    bpt = -(-e // (_TILES * _EB))
    if bpt % _CH:
        bpt += _CH - bpt % _CH
    nch = bpt // _CH
    rows_pad = _TILES * bpt
    grid = rows_pad // _IB
    et_v = edge_type.reshape(rows, 128).astype(jnp.int32)
    srco, srowo = pl.pallas_call(
        _make_idx_body(e),
        grid=(grid,),
        in_specs=[
            pl.BlockSpec((2, _IB * 128), lambda i: (0, i)),
            pl.BlockSpec((_IB, 128), lambda i: (i, 0)),
        ],
        out_specs=[
            pl.BlockSpec((_IB, 128), lambda i: (i, 0)),
            pl.BlockSpec((_IB, 128), lambda i: (i, 0)),
        ],
        out_shape=[
            jax.ShapeDtypeStruct((rows_pad, 128), jnp.int32),
            jax.ShapeDtypeStruct((rows_pad, 128), jnp.int32),
        ],
    )(edge_index, et_v)
    src3 = srco.reshape(_TILES, nch, _CH, _EB)
    srow3 = srowo.reshape(_TILES, nch, _CH, _EB)
    return src3, srow3


def _make_conv_body(relu, bf_out):
    def body(h_ref, a_ref, c0_ref, c1_ref, w_ref, root_ref, bias_ref,
             *out_refs):
        acc = lax.dot_general(h_ref[...], root_ref[...],
                              (((1,), (0,)), ((), ())),
                              preferred_element_type=jnp.float32)
        rc = 1.0 / jnp.maximum(c0_ref[...] + c1_ref[...], 1.0)
        rcx = jnp.concatenate(
            [jnp.broadcast_to(rc[:, r:r + 1], (_BN, _Q)) for r in range(_R)],
            axis=1)
        cat = jnp.concatenate(
            [a_ref[q].astype(jnp.float32) * rcx for q in range(4)], axis=1)
        acc = acc + lax.dot_general(cat, w_ref[...],
                                    (((1,), (0,)), ((), ())),
                                    preferred_element_type=jnp.float32)
        acc = acc + bias_ref[...]
        if relu:
            acc = jnp.maximum(acc, 0.0)
        out_refs[0][...] = acc
        if bf_out:
            out_refs[1][...] = acc.astype(jnp.bfloat16)
    return body


def _conv_combine(h, a, cnt0, cnt1, w, root, bias, relu, bf_out):
    out_specs = [pl.BlockSpec((_BN, _D), lambda i: (i, 0))]
    out_shape = [jax.ShapeDtypeStruct((_N, _D), jnp.float32)]
    if bf_out:
        out_specs.append(pl.BlockSpec((_BN, _D), lambda i: (i, 0)))
        out_shape.append(jax.ShapeDtypeStruct((_N, _D), jnp.bfloat16))
    return pl.pallas_call(
        _make_conv_body(relu, bf_out),
        grid=(_N // _BN,),
        in_specs=[
            pl.BlockSpec((_BN, _D), lambda i: (i, 0)),
            pl.BlockSpec((4, _BN, _D), lambda i: (0, i, 0)),
            pl.BlockSpec((_BN, _R), lambda i: (i, 0)),
            pl.BlockSpec((_BN, _R), lambda i: (i, 0)),
            pl.BlockSpec((4 * _D, _D), lambda i: (0, 0)),
            pl.BlockSpec((_D, _D), lambda i: (0, 0)),
            pl.BlockSpec((1, _D), lambda i: (0, 0)),
        ],
        out_specs=out_specs,
        out_shape=out_shape,
    )(h, a, cnt0, cnt1, w, root, bias.reshape(1, _D))


_CH = 32
_NP = 10240


def _make_edge_body(nch, with_counts):
    def body(hs_ref, src_ref, srow_ref, agg_ref, *rest):
        if with_counts:
            cnt0_ref, cnt1_ref, rest = rest[0], rest[1], rest[2:]
        agg_sh, cnt_sh, table_sh, zbuf = rest[:4]
        rows_bufs = rest[4:4 + _NBUF]
        sidx, ridx, ones_v, rbuf, cbuf, fb0, fb1 = rest[4 + _NBUF:11 + _NBUF]
        fbufs = (fb0, fb1)
        gsems = rest[11 + _NBUF:11 + 2 * _NBUF]
        ssems = rest[11 + 2 * _NBUF:11 + 3 * _NBUF]
        csem = rest[11 + 3 * _NBUF]
        dsems = rest[12 + 3 * _NBUF:14 + 3 * _NBUF]
        cid = lax.axis_index("c")
        sid = lax.axis_index("s")

        z16 = jnp.zeros((16,), jnp.float32)
        z32 = jnp.zeros((32,), jnp.bfloat16)

        def _zero_zbuf(i, _):
            zbuf[i, :] = z32
            return 0
        lax.fori_loop(0, _ZR, _zero_zbuf, 0)

        def _zero_rbuf(i, _):
            rbuf[pl.ds(i * 16, 16)] = z16
            return 0
        lax.fori_loop(0, _STRIPE // 16, _zero_rbuf, 0)

        if with_counts:
            one16 = jnp.ones((16,), jnp.float32)
            for i in range(_EB // 16):
                ones_v[pl.ds(i * 16, 16)] = one16
            pltpu.sync_copy(rbuf, cnt_sh.at[pl.ds(sid * _STRIPE, _STRIPE)])

        ts0 = 632
        ts15 = _N - 15 * ts0
        half = nch // 2
        for p in range(2):
            q = cid + 2 * p

            @pl.when(sid != 15)
            def _():
                pltpu.async_copy(
                    hs_ref.at[pl.ds(sid * ts0, ts0), pl.ds(q * _Q, _Q)],
                    table_sh.at[pl.ds(sid * ts0, ts0), :], csem)

            @pl.when(sid == 15)
            def _():
                pltpu.async_copy(
                    hs_ref.at[pl.ds(15 * ts0, ts15), pl.ds(q * _Q, _Q)],
                    table_sh.at[pl.ds(15 * ts0, ts15), :], csem)

            nz = _STRIPE // _ZR
            for j in range(nz):
                k = j % _NBUF
                if j >= _NBUF:
                    pltpu.make_async_copy(
                        zbuf, agg_sh.at[pl.ds(0, _ZR), :], gsems[k]).wait()
                pltpu.async_copy(
                    zbuf, agg_sh.at[pl.ds(sid * _STRIPE + j * _ZR, _ZR), :],
                    gsems[k])
            for k in range(_NBUF):
                pltpu.make_async_copy(zbuf, agg_sh.at[pl.ds(0, _ZR), :],
                                      gsems[k]).wait()

            @pl.when(sid != 15)
            def _():
                pltpu.make_async_copy(
                    hs_ref.at[pl.ds(sid * ts0, ts0), pl.ds(q * _Q, _Q)],
                    table_sh.at[pl.ds(sid * ts0, ts0), :], csem).wait()

            @pl.when(sid == 15)
            def _():
                pltpu.make_async_copy(
                    hs_ref.at[pl.ds(15 * ts0, ts15), pl.ds(q * _Q, _Q)],
                    table_sh.at[pl.ds(15 * ts0, ts15), :], csem).wait()
            plsc.subcore_barrier()

            counting = with_counts and p == 0

            def _chunk(c, _):
                pltpu.sync_copy(src_ref.at[sid, c], sidx)
                pltpu.sync_copy(srow_ref.at[sid, c], ridx)
                for k in range(_PF):
                    pltpu.async_copy(table_sh.at[sidx.at[k]], rows_bufs[k],
                                     gsems[k])
                for j in range(_CH):
                    k = j % _NBUF
                    if j + _PF < _CH:
                        k2 = (j + _PF) % _NBUF
                        if j + _PF - _NBUF >= 0:
                            pltpu.make_async_copy(
                                rows_bufs[k2], agg_sh.at[ridx.at[0]],
                                ssems[k2]).wait()
                        pltpu.async_copy(table_sh.at[sidx.at[j + _PF]],
                                         rows_bufs[k2], gsems[k2])
                    pltpu.make_async_copy(table_sh.at[sidx.at[0]],
                                          rows_bufs[k], gsems[k]).wait()
                    pltpu.async_copy(rows_bufs[k], agg_sh.at[ridx.at[j]],
                                     ssems[k], add=True)
                    if counting:
                        @pl.when((cid == 0) == (c < half))
                        def _():
                            pltpu.async_copy(ones_v, cnt_sh.at[ridx.at[j]],
                                             csem, add=True)
                for j in range(max(_CH - _NBUF, 0), _CH):
                    k = j % _NBUF
                    pltpu.make_async_copy(rows_bufs[k], agg_sh.at[ridx.at[0]],
                                          ssems[k]).wait()
                if counting:
                    @pl.when((cid == 0) == (c < half))
                    def _():
                        for j in range(_CH):
                            pltpu.make_async_copy(
                                ones_v, cnt_sh.at[ridx.at[0]], csem).wait()
                return 0
            lax.fori_loop(0, nch, _chunk, 0)
            plsc.subcore_barrier()

            iota2 = lax.iota(jnp.int32, 16) * 2
            for t in range(_STRIPE // _EB):
                pltpu.sync_copy(
                    agg_sh.at[pl.ds(sid * _STRIPE + t * _EB, _EB), :], cbuf)
                fb = fbufs[t % 2]
                dst = agg_ref.at[
                    pl.ds(q * _RN_PAD + sid * _STRIPE + t * _EB, _EB), :]
                if t >= 2:
                    prev = agg_ref.at[
                        pl.ds(q * _RN_PAD + sid * _STRIPE + (t - 2) * _EB,
                              _EB), :]
                    pltpu.make_async_copy(fb, prev, dsems[t % 2]).wait()

                def _cv(i, _):
                    ci = plsc.bitcast(cbuf[i, :], jnp.int32)
                    fe = plsc.bitcast(ci << 16, jnp.float32)
                    fo = plsc.bitcast(ci & jnp.int32(-65536), jnp.float32)
                    rowi = jnp.full((16,), i, jnp.int32)
                    plsc.store_scatter(fb, [rowi, iota2], fe)
                    plsc.store_scatter(fb, [rowi, iota2 + 1], fo)
                    return 0
                lax.fori_loop(0, _EB, _cv, 0)
                pltpu.async_copy(fb, dst, dsems[t % 2])
            for t in range(_STRIPE // _EB - 2, _STRIPE // _EB):
                fin = agg_ref.at[
                    pl.ds(q * _RN_PAD + sid * _STRIPE + t * _EB, _EB), :]
                pltpu.make_async_copy(fbufs[t % 2], fin, dsems[t % 2]).wait()
            if with_counts and p == 0:
                @pl.when(cid == 0)
                def _():
                    pltpu.sync_copy(cnt_sh.at[pl.ds(sid * _STRIPE, _STRIPE)],
                                    cnt0_ref.at[pl.ds(sid * _STRIPE,
                                                      _STRIPE)])

                @pl.when(cid == 1)
                def _():
                    pltpu.sync_copy(cnt_sh.at[pl.ds(sid * _STRIPE, _STRIPE)],
                                    cnt1_ref.at[pl.ds(sid * _STRIPE,
                                                      _STRIPE)])
            plsc.subcore_barrier()

    return body


def _edge_pass(hs, src4, srow3, with_counts):
    nch = src4.shape[1]
    out_type = [jax.ShapeDtypeStruct((4 * _RN_PAD, _Q), jnp.float32)]
    if with_counts:
        out_type.append(jax.ShapeDtypeStruct((_RN_PAD,), jnp.float32))
        out_type.append(jax.ShapeDtypeStruct((_RN_PAD,), jnp.float32))
    scratch = (
        [
            pltpu.VMEM_SHARED((_RN_PAD, _Q), jnp.bfloat16),
            pltpu.VMEM_SHARED((_RN_PAD,), jnp.float32),
            pltpu.VMEM_SHARED((_NP, _Q), jnp.bfloat16),
            pltpu.VMEM((_ZR, _Q), jnp.bfloat16),
        ]
        + [pltpu.VMEM((_EB, _Q), jnp.bfloat16)] * _NBUF
        + [
            pltpu.VMEM((_CH, _EB), jnp.int32),
            pltpu.VMEM((_CH, _EB), jnp.int32),
            pltpu.VMEM((_EB,), jnp.float32),
            pltpu.VMEM((_STRIPE,), jnp.float32),
            pltpu.VMEM((_EB, _Q), jnp.bfloat16),
            pltpu.VMEM((_EB, _Q), jnp.float32),
            pltpu.VMEM((_EB, _Q), jnp.float32),
        ]
        + [pltpu.SemaphoreType.DMA] * (2 * _NBUF + 3)
    )
    mesh = plsc.VectorSubcoreMesh(core_axis_name="c", subcore_axis_name="s",
                                  num_cores=2, num_subcores=_TILES)
    fn = pl.kernel(
        _make_edge_body(nch, with_counts),
        out_type=tuple(out_type),
        mesh=mesh,
        scratch_types=scratch,
        compiler_params=pltpu.CompilerParams(use_tc_tiling_on_sc=False,
                                             needs_layout_passes=False),
    )
    return fn(hs, src4, srow3)


def kernel(x, W_proj, b_proj, basis0, comp0, root0, bias0,
           basis1, comp1, root1, bias1, edge_index, edge_type):
    w0, w1 = _combine_weights(comp0, basis0, comp1, basis1)
    h, hb = _project(x, W_proj, b_proj)
    src3, srow3 = _edge_indices(edge_index, edge_type)

    agg0, cnt0, cnt1 = _edge_pass(hb, src3, srow3, with_counts=True)
    a0 = agg0.reshape(4, _RN_PAD // _R, _D)
    c0 = cnt0.reshape(_RN_PAD // _R, _R)
    c1 = cnt1.reshape(_RN_PAD // _R, _R)
    x1, x1b = _conv_combine(h, a0, c0, c1, w0, root0, bias0,
                            relu=True, bf_out=True)

    agg1 = _edge_pass(x1b, src3, srow3, with_counts=False)[0]
    a1 = agg1.reshape(4, _RN_PAD // _R, _D)
    out = _conv_combine(x1, a1, c0, c1, w1, root1, bias1,
                        relu=False, bf_out=False)[0]
    return out, h, h
